# Initial kernel scaffold; baseline (speedup 1.0000x reference)
import jax
import jax.numpy as jnp
from jax.experimental import pallas as pl


def kernel(u, v, w, n, edge_index_p, edge_index_n, edge_index_n1, E_pos, E_neg, E_item, E_item_n, E2, W0, b0, W1, b1, Wa, ba, Wq):
    raise NotImplementedError("write your pallas kernel here")


# speedup vs baseline: 18.2084x; 18.2084x over previous
import functools

import jax
import jax.numpy as jnp
from jax import lax
from jax.experimental import pallas as pl
from jax.experimental.pallas import tpu as pltpu
from jax.experimental.pallas import tpu_sc as plsc

_M = 30000
_NI = 20000
_N = 50000
_NP = 51200
_D = 64
_HALF = 32
_NE = 800000
_NEP = 819200
_B = 4096
_K = 10
_REG = 1e-4
_TAU = 0.8

_NW = 32
_CH = 128
_NCHUNK = (_NEP // _NW) // _CH
_RPT = _NP // 16
_ZR = 100
_CR = _NP // _CH
_MCH = 80

_RB = 512
_GRID = _NP // _RB


def _mesh():
    return plsc.VectorSubcoreMesh(core_axis_name="c", subcore_axis_name="s")


def _sc_params():
    return pltpu.CompilerParams(needs_layout_passes=False,
                                use_tc_tiling_on_sc=False)


_IDXROWS = 40


def _sc_degrees(idx6):
    @functools.partial(
        pl.kernel,
        out_type=jax.ShapeDtypeStruct((6 * _NW * _NP,), jnp.float32),
        mesh=_mesh(),
        compiler_params=_sc_params(),
        scratch_types=[
            pltpu.VMEM((_NP,), jnp.float32),
            pltpu.VMEM((_IDXROWS, _CH), jnp.int32),
        ],
    )
    def deg_kernel(idx_hbm, out_hbm, counts_v, idxbuf_v):
        c = lax.axis_index("c")
        sid = lax.axis_index("s")
        wid = sid * 2 + c
        zero16 = jnp.zeros((16,), jnp.float32)
        ones16 = jnp.ones((16,), jnp.float32)

        for s in range(6):
            def zb(i, carry):
                counts_v[pl.ds(i * 16, 16)] = zero16
                return carry
            lax.fori_loop(0, _NP // 16, zb, None, unroll=4)

            for blk in range(_NCHUNK // _IDXROWS):
                pltpu.sync_copy(
                    idx_hbm.at[s, pl.ds(wid * _NCHUNK + blk * _IDXROWS,
                                        _IDXROWS)],
                    idxbuf_v)

                def chunk_body(j, carry):
                    for g in range(_CH // 16):
                        iv = idxbuf_v[j, pl.ds(g * 16, 16)]
                        plsc.addupdate_scatter(counts_v, [iv], ones16)
                    return carry

                lax.fori_loop(0, _IDXROWS, chunk_body, None)

            pltpu.sync_copy(counts_v,
                            out_hbm.at[pl.ds((s * _NW + wid) * _NP, _NP)])

    return deg_kernel(idx6)


def _sc_gconv(tabs, srcs, dsts):
    half = jax.ShapeDtypeStruct((_NP, _HALF), jnp.float32)

    @functools.partial(
        pl.kernel,
        out_type=[half] * 6,
        mesh=_mesh(),
        compiler_params=_sc_params(),
        scratch_types=[
            pltpu.VMEM((_IDXROWS, _CH), jnp.int32),
            pltpu.VMEM((_IDXROWS, _CH), jnp.int32),
            pltpu.VMEM((_CH, _HALF), jnp.float32),
            pltpu.VMEM((_ZR, _HALF), jnp.float32),
            pltpu.SemaphoreType.DMA,
            pltpu.VMEM_SHARED((_NP, _HALF), jnp.float32),
        ],
    )
    def gconv_kernel(tA0, tB0, tA1, tB1, tA2, tB2, src_hbm, dst_hbm,
                     oA0, oB0, oA1, oB1, oA2, oB2,
                     srcbuf, dstbuf, rows_v, zbuf, sem, acc):
        c = lax.axis_index("c")
        sid = lax.axis_index("s")
        wid = sid * 2 + c
        zero16 = jnp.zeros((16,), jnp.float32)

        def zb(i, carry):
            zbuf[i, pl.ds(0, 16)] = zero16
            zbuf[i, pl.ds(16, 16)] = zero16
            return carry
        lax.fori_loop(0, _ZR, zb, None, unroll=4)

        groups = ((tA0, oA0, tB0, oB0), (tA1, oA1, tB1, oB1),
                  (tA2, oA2, tB2, oB2))
        for g in range(3):
            tA, oA, tB, oB = groups[g]
            for r in range(_RPT // _ZR):
                pltpu.sync_copy(zbuf, acc.at[pl.ds(sid * _RPT + r * _ZR, _ZR)])
            plsc.subcore_barrier()

            for blk in range(_NCHUNK // _IDXROWS):
                base = wid * _NCHUNK + blk * _IDXROWS
                pltpu.sync_copy(src_hbm.at[g, pl.ds(base, _IDXROWS)], srcbuf)
                pltpu.sync_copy(dst_hbm.at[g, pl.ds(base, _IDXROWS)], dstbuf)

                def chunk_body(j, carry):
                    @pl.when(c == 0)
                    def _():
                        pltpu.async_copy(tA.at[srcbuf.at[j]], rows_v,
                                         sem).wait()

                    @pl.when(c == 1)
                    def _():
                        pltpu.async_copy(tB.at[srcbuf.at[j]], rows_v,
                                         sem).wait()

                    pltpu.sync_copy(rows_v, acc.at[dstbuf.at[j]], add=True)
                    return carry

                lax.fori_loop(0, _IDXROWS, chunk_body, None)
            plsc.subcore_barrier()

            @pl.when(c == 0)
            def _():
                pltpu.sync_copy(acc.at[pl.ds(sid * _RPT, _RPT)],
                                oA.at[pl.ds(sid * _RPT, _RPT)])

            @pl.when(c == 1)
            def _():
                pltpu.sync_copy(acc.at[pl.ds(sid * _RPT, _RPT)],
                                oB.at[pl.ds(sid * _RPT, _RPT)])

    return gconv_kernel(*tabs, srcs, dsts)


def _sc_batch_gather(zc, sn, sn1, u2d, v2d, n2d):
    full = jax.ShapeDtypeStruct((_B, _D), jnp.float32)
    nfull = jax.ShapeDtypeStruct((_B * _K, _D), jnp.float32)

    @functools.partial(
        pl.kernel,
        out_type=[full, full, nfull, full, full, nfull, full, full],
        mesh=_mesh(),
        compiler_params=_sc_params(),
        scratch_types=[
            pltpu.VMEM((_NW, _CH), jnp.int32),
            pltpu.VMEM((_NW, _CH), jnp.int32),
            pltpu.VMEM((_NW * _K, _CH), jnp.int32),
            pltpu.VMEM((_CH, _D), jnp.float32),
            pltpu.SemaphoreType.DMA,
        ],
    )
    def gather_kernel(zc_hbm, sn_hbm, sn1_hbm, u_hbm, v_hbm, n_hbm,
                      ou1, ov1, on1, ou2, ov2, on2, ou3, ov3,
                      ubuf, vbuf, nbuf, rows_v, sem):
        c = lax.axis_index("c")
        sid = lax.axis_index("s")
        wid = sid * 2 + c

        pltpu.sync_copy(u_hbm, ubuf)
        pltpu.sync_copy(v_hbm, vbuf)
        pltpu.sync_copy(n_hbm, nbuf)

        for tab, ibuf, out in ((zc_hbm, ubuf, ou1), (zc_hbm, vbuf, ov1),
                               (sn_hbm, ubuf, ou2), (sn_hbm, vbuf, ov2),
                               (sn1_hbm, ubuf, ou3), (sn1_hbm, vbuf, ov3)):
            pltpu.async_copy(tab.at[ibuf.at[wid]], rows_v, sem).wait()
            pltpu.sync_copy(rows_v, out.at[pl.ds(wid * _CH, _CH)])

        for tab, out in ((zc_hbm, on1), (sn_hbm, on2)):
            for r in range(_K):
                pltpu.async_copy(tab.at[nbuf.at[wid * _K + r]],
                                 rows_v, sem).wait()
                pltpu.sync_copy(rows_v,
                                out.at[pl.ds(wid * _K * _CH + r * _CH, _CH)])

    return gather_kernel(zc, sn, sn1, u2d, v2d, n2d)


def _isd_spec():
    return pl.BlockSpec((6, _RB), lambda i: (0, i))


def _half_spec():
    return pl.BlockSpec((_RB, _HALF), lambda i: (i, 0))


def _full_spec():
    return pl.BlockSpec((_RB, _D), lambda i: (i, 0))


def _tc_prep(dpart, x0p, x0n):
    half = jax.ShapeDtypeStruct((_NP, _HALF), jnp.float32)
    isd6 = jax.ShapeDtypeStruct((6, _NP), jnp.float32)

    def body(dp_ref, xp_ref, xn_ref, opA, opB, onA, onB, om1A, om1B, oisd):
        deg = jnp.sum(dp_ref[...], axis=1)
        isd = lax.rsqrt(jnp.maximum(deg, 1.0))
        oisd[...] = isd
        xp = xp_ref[...]
        xn = xn_ref[...]
        yp = xp * isd[0][:, None]
        opA[...] = yp[:, :_HALF]
        opB[...] = yp[:, _HALF:]
        yn = xn * isd[2][:, None]
        onA[...] = yn[:, :_HALF]
        onB[...] = yn[:, _HALF:]
        ym = xn * isd[4][:, None]
        om1A[...] = ym[:, :_HALF]
        om1B[...] = ym[:, _HALF:]

    return pl.pallas_call(
        body,
        grid=(_GRID,),
        in_specs=[pl.BlockSpec((6, _NW, _RB), lambda i: (0, 0, i)),
                  _full_spec(), _full_spec()],
        out_specs=[_half_spec()] * 6 + [_isd_spec()],
        out_shape=[half] * 6 + [isd6],
    )(dpart, x0p, x0n)


def _tc_inter(isd6, acc1):
    half = jax.ShapeDtypeStruct((_NP, _HALF), jnp.float32)

    def body(isd_ref, aA0, aB0, aA1, aB1, aA2, aB2,
             oA0, oB0, oA1, oB1, oA2, oB2):
        isd = isd_ref[...]
        groups = ((aA0, aB0, oA0, oB0), (aA1, aB1, oA1, oB1),
                  (aA2, aB2, oA2, oB2))
        for g in range(3):
            aA, aB, oA, oB = groups[g]
            wsd = (isd[2 * g] * isd[2 * g + 1])[:, None]
            oA[...] = aA[...] * wsd
            oB[...] = aB[...] * wsd

    return pl.pallas_call(
        body,
        grid=(_GRID,),
        in_specs=[_isd_spec()] + [_half_spec()] * 6,
        out_specs=[_half_spec()] * 6,
        out_shape=[half] * 6,
    )(isd6, *acc1)


def _dot_t(x, wref):
    return lax.dot_general(x, wref[...], (((1,), (1,)), ((), ())),
                           preferred_element_type=jnp.float32)


def _tc_combine(isd6, x0p, x0n, e2, acc1, acc2, W0, b0, W1, b1, Wa, ba, Wq):
    full = jax.ShapeDtypeStruct((_NP, _D), jnp.float32)
    wspec = pl.BlockSpec((_D, _D), lambda i: (0, 0))
    bspec = pl.BlockSpec((1, _D), lambda i: (0, 0))
    qspec = pl.BlockSpec((1, _D), lambda i: (0, 0))

    def body(isd_ref, xp_ref, xn_ref, e2_ref,
             a1pA, a1pB, a1nA, a1nB, a1mA, a1mB,
             a2pA, a2pB, a2nA, a2nB, a2mA, a2mB,
             W0r, b0r, W1r, b1r, War, bar, Wqr,
             ozc, osn, osn1):
        isd = isd_ref[...]

        def comb(x0, aA1, aB1, aA2, aB2, s):
            accsum = jnp.concatenate(
                [aA1[...] + aA2[...], aB1[...] + aB2[...]], axis=1)
            return (x0 + isd[s][:, None] * accsum) * (1.0 / 3.0)

        sp = comb(xp_ref[...], a1pA, a1pB, a2pA, a2pB, 1)
        sn = comb(xn_ref[...], a1nA, a1nB, a2nA, a2nB, 3)
        sn1 = comb(xn_ref[...], a1mA, a1mB, a2mA, a2mB, 5)

        h = jnp.maximum(_dot_t(e2_ref[...], W0r) + b0r[...], 0.0)
        zng = jnp.maximum(_dot_t(h, W1r) + b1r[...], 0.0)

        wp = _dot_t(jnp.tanh(_dot_t(sp, War) + bar[...]), Wqr)
        wn = _dot_t(jnp.tanh(_dot_t(zng, War) + bar[...]), Wqr)
        m = jnp.maximum(wp, wn)
        e0 = jnp.exp(wp - m)
        e1 = jnp.exp(wn - m)
        a0 = e0 / (e0 + e1)
        ozc[...] = a0 * sp + (1.0 - a0) * zng
        osn[...] = sn
        osn1[...] = sn1

    return pl.pallas_call(
        body,
        grid=(_GRID,),
        in_specs=([_isd_spec(), _full_spec(), _full_spec(), _full_spec()]
                  + [_half_spec()] * 12
                  + [wspec, bspec, wspec, bspec, wspec, bspec, qspec]),
        out_specs=[_full_spec()] * 3,
        out_shape=[full] * 3,
    )(isd6, x0p, x0n, e2, *acc1, *acc2, W0, b0, W1, b1, Wa, ba, Wq)


def _log_sigmoid(x):
    return jnp.minimum(x, 0.0) - jnp.log1p(jnp.exp(-jnp.abs(x)))


def _rnorm(a):
    return a * lax.rsqrt(jnp.maximum(jnp.sum(a * a, axis=1, keepdims=True),
                                     1e-24))


def _tc_loss(u1, v1, u2, v2, u3, v3, n1, n2, wsg):
    RB2 = 512
    NB2 = _B // RB2
    bspec = pl.BlockSpec((RB2, _D), lambda i: (i, 0))
    nspec = pl.BlockSpec((RB2, _K, _D), lambda i: (i, 0, 0))
    wspec = pl.BlockSpec((RB2, 1), lambda i: (i, 0))
    fullspec = pl.BlockSpec((_B, _D), lambda i: (0, 0))

    def body(u1r, v1r, u2r, v2r, u3r, v3r, n1r, n2r, wr, u3f, v3f,
             out, diag_u, diag_v, accs):
        i = pl.program_id(0)

        @pl.when(i == 0)
        def _():
            for k in range(6):
                accs[k] = 0.0

        u1b = u1r[...]
        v1b = v1r[...]
        u2b = u2r[...]
        v2b = v2r[...]
        n1b = n1r[...]
        n2b = n2r[...]
        sgn = jnp.sign(wr[...])

        pos1 = jnp.sum(u1b * v1b, axis=1, keepdims=True)
        neg1 = jnp.sum(u1b[:, None, :] * n1b, axis=2)
        sb1 = jnp.sum(_log_sigmoid((-sgn + 2.0) * pos1 - neg1))
        r1 = jnp.sum(u1b * u1b) + jnp.sum(v1b * v1b) + jnp.sum(n1b * n1b)

        pos2 = jnp.sum(u2b * v2b, axis=1, keepdims=True)
        neg2 = jnp.sum(u2b[:, None, :] * n2b, axis=2)
        sb2 = jnp.sum(_log_sigmoid(neg2 - (sgn + 2.0) * pos2))
        r2 = jnp.sum(u2b * u2b) + jnp.sum(v2b * v2b) + jnp.sum(n2b * n2b)

        u2n = _rnorm(u2b)
        v2n = _rnorm(v2b)
        u3n_blk = _rnorm(u3r[...])
        v3n_blk = _rnorm(v3r[...])
        u3n_all = _rnorm(u3f[...])
        v3n_all = _rnorm(v3f[...])
        fu = jnp.exp(lax.dot_general(u2n, u3n_all, (((1,), (1,)), ((), ())),
                                     preferred_element_type=jnp.float32)
                     / _TAU)
        fv = jnp.exp(lax.dot_general(v2n, v3n_all, (((1,), (1,)), ((), ())),
                                     preferred_element_type=jnp.float32)
                     / _TAU)
        du = jnp.exp(jnp.sum(u2n * u3n_blk, axis=1, keepdims=True) / _TAU)
        dv = jnp.exp(jnp.sum(v2n * v3n_blk, axis=1, keepdims=True) / _TAU)
        diag_u[pl.ds(i * RB2, RB2), :] = du
        diag_v[pl.ds(i * RB2, RB2), :] = dv

        accs[0] = accs[0] + sb1
        accs[1] = accs[1] + r1
        accs[2] = accs[2] + sb2
        accs[3] = accs[3] + r2
        accs[4] = accs[4] + jnp.sum(fu)
        accs[5] = accs[5] + jnp.sum(fv)

        @pl.when(i == NB2 - 1)
        def _():
            du_all = diag_u[...]
            dv_all = diag_v[...]
            pos = du_all + dv_all
            neg = (accs[4] + accs[5]) - du_all - dv_all
            cl = -jnp.log(pos / (pos + neg))
            cl_mean = jnp.sum(cl) / float(_B)
            loss = (-accs[0] + _REG * accs[1]
                    + (-accs[2] / float(_B) + _REG * accs[3])
                    + cl_mean)
            out[0, 0] = loss

    return pl.pallas_call(
        body,
        grid=(NB2,),
        in_specs=[bspec] * 6 + [nspec, nspec, wspec, fullspec, fullspec],
        out_specs=pl.BlockSpec(memory_space=pltpu.SMEM),
        out_shape=jax.ShapeDtypeStruct((1, 1), jnp.float32),
        scratch_shapes=[
            pltpu.VMEM((_B, 1), jnp.float32),
            pltpu.VMEM((_B, 1), jnp.float32),
            pltpu.SMEM((8,), jnp.float32),
        ],
    )(u1, v1, u2, v2, u3, v3, n1, n2, wsg, u3, v3)


def kernel(u, v, w, n, edge_index_p, edge_index_n, edge_index_n1,
           E_pos, E_neg, E_item, E_item_n, E2,
           W0, b0, W1, b1, Wa, ba, Wq):
    f32 = jnp.float32
    i32 = jnp.int32
    npad = _NP - _N
    pe = _NEP - _NE
    fill = (_N + (jnp.arange(pe, dtype=i32) % npad)).astype(i32)

    def pad_edges(ei):
        src = jnp.concatenate([ei[0].astype(i32), fill])
        dst = jnp.concatenate([ei[1].astype(i32), fill])
        return (src.reshape(_NEP // _CH, _CH), dst.reshape(_NEP // _CH, _CH))

    sp_, dp_ = pad_edges(edge_index_p)
    sn_, dn_ = pad_edges(edge_index_n)
    sm_, dm_ = pad_edges(edge_index_n1)
    idx6 = jnp.stack([sp_, dp_, sn_, dn_, sm_, dm_])

    dpart = _sc_degrees(idx6).reshape(6, _NW, _NP)

    zrows = jnp.zeros((npad, _D), f32)
    x0p = jnp.concatenate([E_pos.astype(f32), E_item.astype(f32), zrows])
    x0n = jnp.concatenate([E_neg.astype(f32), E_item_n.astype(f32), zrows])

    prep = _tc_prep(dpart, x0p, x0n)
    y0 = prep[:6]
    isd6 = prep[6]
    srcs = jnp.stack([sp_, sn_, sm_])
    dsts = jnp.stack([dp_, dn_, dm_])
    acc1 = _sc_gconv(y0, srcs, dsts)
    y1 = _tc_inter(isd6, acc1)
    acc2 = _sc_gconv(y1, srcs, dsts)

    e2p = jnp.concatenate([E2.astype(f32), zrows])
    zc, snt, sn1t = _tc_combine(isd6, x0p, x0n, e2p, acc1, acc2,
                                W0, b0.reshape(1, _D), W1, b1.reshape(1, _D),
                                Wa, ba.reshape(1, _D), Wq)

    u2d = u.astype(i32).reshape(_NW, _CH)
    v2d = v.astype(i32).reshape(_NW, _CH)
    n2d = n.astype(i32).reshape((_B * _K) // _CH, _CH)
    g = _sc_batch_gather(zc, snt, sn1t, u2d, v2d, n2d)
    u1g, v1g, n1g, u2g, v2g, n2g, u3g, v3g = g

    loss = _tc_loss(u1g, v1g, u2g, v2g, u3g, v3g,
                    n1g.reshape(_B, _K, _D), n2g.reshape(_B, _K, _D),
                    w.astype(f32).reshape(_B, 1))
    return loss[0, 0]


# speedup vs baseline: 19.9324x; 1.0947x over previous
import functools

import jax
import jax.numpy as jnp
from jax import lax
from jax.experimental import pallas as pl
from jax.experimental.pallas import tpu as pltpu
from jax.experimental.pallas import tpu_sc as plsc

_M = 30000
_NI = 20000
_N = 50000
_NP = 51200
_D = 64
_HALF = 32
_NE = 800000
_NEP = 819200
_B = 4096
_K = 10
_REG = 1e-4
_TAU = 0.8

_NW = 32
_CH = 128
_NCHUNK = (_NEP // _NW) // _CH
_RPT = _NP // 16
_ZR = 100
_CR = _NP // _CH
_MCH = 80

_RB = 512
_GRID = _NP // _RB


def _mesh():
    return plsc.VectorSubcoreMesh(core_axis_name="c", subcore_axis_name="s")


def _sc_params():
    return pltpu.CompilerParams(needs_layout_passes=False,
                                use_tc_tiling_on_sc=False)


_IDXROWS = 40


def _sc_degrees(idx6):
    @functools.partial(
        pl.kernel,
        out_type=jax.ShapeDtypeStruct((6 * _NW * _NP,), jnp.float32),
        mesh=_mesh(),
        compiler_params=_sc_params(),
        scratch_types=[
            pltpu.VMEM((_NP,), jnp.float32),
            pltpu.VMEM((_IDXROWS, _CH), jnp.int32),
        ],
    )
    def deg_kernel(idx_hbm, out_hbm, counts_v, idxbuf_v):
        c = lax.axis_index("c")
        sid = lax.axis_index("s")
        wid = sid * 2 + c
        zero16 = jnp.zeros((16,), jnp.float32)
        ones16 = jnp.ones((16,), jnp.float32)

        for s in range(6):
            def zb(i, carry):
                counts_v[pl.ds(i * 16, 16)] = zero16
                return carry
            lax.fori_loop(0, _NP // 16, zb, None, unroll=4)

            for blk in range(_NCHUNK // _IDXROWS):
                pltpu.sync_copy(
                    idx_hbm.at[s, pl.ds(wid * _NCHUNK + blk * _IDXROWS,
                                        _IDXROWS)],
                    idxbuf_v)

                def chunk_body(j, carry):
                    for g in range(_CH // 16):
                        iv = idxbuf_v[j, pl.ds(g * 16, 16)]
                        plsc.addupdate_scatter(counts_v, [iv], ones16)
                    return carry

                lax.fori_loop(0, _IDXROWS, chunk_body, None)

            pltpu.sync_copy(counts_v,
                            out_hbm.at[pl.ds((s * _NW + wid) * _NP, _NP)])

    return deg_kernel(idx6)


def _sc_gconv(tabs, srcs, dsts):
    half = jax.ShapeDtypeStruct((_NP, _HALF), jnp.float32)

    @functools.partial(
        pl.kernel,
        out_type=[half] * 6,
        mesh=_mesh(),
        compiler_params=_sc_params(),
        scratch_types=[
            pltpu.VMEM((_IDXROWS, _CH), jnp.int32),
            pltpu.VMEM((_IDXROWS, _CH), jnp.int32),
            pltpu.VMEM((_CH, _HALF), jnp.float32),
            pltpu.VMEM((_CH, _HALF), jnp.float32),
            pltpu.VMEM((_ZR, _HALF), jnp.float32),
            pltpu.SemaphoreType.DMA,
            pltpu.SemaphoreType.DMA,
            pltpu.SemaphoreType.DMA,
            pltpu.SemaphoreType.DMA,
            pltpu.VMEM_SHARED((_NP, _HALF), jnp.float32),
        ],
    )
    def gconv_kernel(tA0, tB0, tA1, tB1, tA2, tB2, src_hbm, dst_hbm,
                     oA0, oB0, oA1, oB1, oA2, oB2,
                     srcbuf, dstbuf, rows0, rows1, zbuf,
                     gsem0, gsem1, ssem0, ssem1, acc):
        c = lax.axis_index("c")
        sid = lax.axis_index("s")
        wid = sid * 2 + c
        zero16 = jnp.zeros((16,), jnp.float32)

        def zb(i, carry):
            zbuf[i, pl.ds(0, 16)] = zero16
            zbuf[i, pl.ds(16, 16)] = zero16
            return carry
        lax.fori_loop(0, _ZR, zb, None, unroll=4)

        groups = ((tA0, oA0, tB0, oB0), (tA1, oA1, tB1, oB1),
                  (tA2, oA2, tB2, oB2))
        for g in range(3):
            tA, oA, tB, oB = groups[g]
            for r in range(_RPT // _ZR):
                pltpu.sync_copy(zbuf, acc.at[pl.ds(sid * _RPT + r * _ZR, _ZR)])
            plsc.subcore_barrier()

            def start_gather(j, buf, sem):
                @pl.when(c == 0)
                def _():
                    pltpu.async_copy(tA.at[srcbuf.at[j]], buf, sem)

                @pl.when(c == 1)
                def _():
                    pltpu.async_copy(tB.at[srcbuf.at[j]], buf, sem)

            def wait_gather(buf, sem):
                pltpu.make_async_copy(tA.at[srcbuf.at[0]], buf, sem).wait()

            def start_scatter(j, buf, sem):
                pltpu.async_copy(buf, acc.at[dstbuf.at[j]], sem, add=True)

            def wait_scatter(buf, sem):
                pltpu.make_async_copy(buf, acc.at[dstbuf.at[0]], sem).wait()

            for blk in range(_NCHUNK // _IDXROWS):
                base = wid * _NCHUNK + blk * _IDXROWS
                pltpu.sync_copy(src_hbm.at[g, pl.ds(base, _IDXROWS)], srcbuf)
                pltpu.sync_copy(dst_hbm.at[g, pl.ds(base, _IDXROWS)], dstbuf)

                start_gather(0, rows0, gsem0)

                def pair_body(t, carry):
                    wait_gather(rows0, gsem0)
                    start_gather(2 * t + 1, rows1, gsem1)
                    start_scatter(2 * t, rows0, ssem0)
                    wait_gather(rows1, gsem1)
                    wait_scatter(rows0, ssem0)

                    @pl.when(t < _IDXROWS // 2 - 1)
                    def _():
                        start_gather(2 * t + 2, rows0, gsem0)

                    start_scatter(2 * t + 1, rows1, ssem1)
                    wait_scatter(rows1, ssem1)
                    return carry

                lax.fori_loop(0, _IDXROWS // 2, pair_body, None)
            plsc.subcore_barrier()

            @pl.when(c == 0)
            def _():
                pltpu.sync_copy(acc.at[pl.ds(sid * _RPT, _RPT)],
                                oA.at[pl.ds(sid * _RPT, _RPT)])

            @pl.when(c == 1)
            def _():
                pltpu.sync_copy(acc.at[pl.ds(sid * _RPT, _RPT)],
                                oB.at[pl.ds(sid * _RPT, _RPT)])

    return gconv_kernel(*tabs, srcs, dsts)


def _sc_batch_gather(zc, sn, sn1, u2d, v2d, n2d):
    full = jax.ShapeDtypeStruct((_B, _D), jnp.float32)
    nfull = jax.ShapeDtypeStruct((_B * _K, _D), jnp.float32)

    @functools.partial(
        pl.kernel,
        out_type=[full, full, nfull, full, full, nfull, full, full],
        mesh=_mesh(),
        compiler_params=_sc_params(),
        scratch_types=[
            pltpu.VMEM((_NW, _CH), jnp.int32),
            pltpu.VMEM((_NW, _CH), jnp.int32),
            pltpu.VMEM((_NW * _K, _CH), jnp.int32),
            pltpu.VMEM((_CH, _D), jnp.float32),
            pltpu.SemaphoreType.DMA,
        ],
    )
    def gather_kernel(zc_hbm, sn_hbm, sn1_hbm, u_hbm, v_hbm, n_hbm,
                      ou1, ov1, on1, ou2, ov2, on2, ou3, ov3,
                      ubuf, vbuf, nbuf, rows_v, sem):
        c = lax.axis_index("c")
        sid = lax.axis_index("s")
        wid = sid * 2 + c

        pltpu.sync_copy(u_hbm, ubuf)
        pltpu.sync_copy(v_hbm, vbuf)
        pltpu.sync_copy(n_hbm, nbuf)

        for tab, ibuf, out in ((zc_hbm, ubuf, ou1), (zc_hbm, vbuf, ov1),
                               (sn_hbm, ubuf, ou2), (sn_hbm, vbuf, ov2),
                               (sn1_hbm, ubuf, ou3), (sn1_hbm, vbuf, ov3)):
            pltpu.async_copy(tab.at[ibuf.at[wid]], rows_v, sem).wait()
            pltpu.sync_copy(rows_v, out.at[pl.ds(wid * _CH, _CH)])

        for tab, out in ((zc_hbm, on1), (sn_hbm, on2)):
            for r in range(_K):
                pltpu.async_copy(tab.at[nbuf.at[wid * _K + r]],
                                 rows_v, sem).wait()
                pltpu.sync_copy(rows_v,
                                out.at[pl.ds(wid * _K * _CH + r * _CH, _CH)])

    return gather_kernel(zc, sn, sn1, u2d, v2d, n2d)


def _isd_spec():
    return pl.BlockSpec((6, _RB), lambda i: (0, i))


def _half_spec():
    return pl.BlockSpec((_RB, _HALF), lambda i: (i, 0))


def _full_spec():
    return pl.BlockSpec((_RB, _D), lambda i: (i, 0))


def _tc_prep(dpart, x0p, x0n):
    half = jax.ShapeDtypeStruct((_NP, _HALF), jnp.float32)
    isd6 = jax.ShapeDtypeStruct((6, _NP), jnp.float32)

    def body(dp_ref, xp_ref, xn_ref, opA, opB, onA, onB, om1A, om1B, oisd):
        deg = jnp.sum(dp_ref[...], axis=1)
        isd = lax.rsqrt(jnp.maximum(deg, 1.0))
        oisd[...] = isd
        xp = xp_ref[...]
        xn = xn_ref[...]
        yp = xp * isd[0][:, None]
        opA[...] = yp[:, :_HALF]
        opB[...] = yp[:, _HALF:]
        yn = xn * isd[2][:, None]
        onA[...] = yn[:, :_HALF]
        onB[...] = yn[:, _HALF:]
        ym = xn * isd[4][:, None]
        om1A[...] = ym[:, :_HALF]
        om1B[...] = ym[:, _HALF:]

    return pl.pallas_call(
        body,
        grid=(_GRID,),
        in_specs=[pl.BlockSpec((6, _NW, _RB), lambda i: (0, 0, i)),
                  _full_spec(), _full_spec()],
        out_specs=[_half_spec()] * 6 + [_isd_spec()],
        out_shape=[half] * 6 + [isd6],
    )(dpart, x0p, x0n)


def _tc_inter(isd6, acc1):
    half = jax.ShapeDtypeStruct((_NP, _HALF), jnp.float32)

    def body(isd_ref, aA0, aB0, aA1, aB1, aA2, aB2,
             oA0, oB0, oA1, oB1, oA2, oB2):
        isd = isd_ref[...]
        groups = ((aA0, aB0, oA0, oB0), (aA1, aB1, oA1, oB1),
                  (aA2, aB2, oA2, oB2))
        for g in range(3):
            aA, aB, oA, oB = groups[g]
            wsd = (isd[2 * g] * isd[2 * g + 1])[:, None]
            oA[...] = aA[...] * wsd
            oB[...] = aB[...] * wsd

    return pl.pallas_call(
        body,
        grid=(_GRID,),
        in_specs=[_isd_spec()] + [_half_spec()] * 6,
        out_specs=[_half_spec()] * 6,
        out_shape=[half] * 6,
    )(isd6, *acc1)


def _dot_t(x, wref):
    return lax.dot_general(x, wref[...], (((1,), (1,)), ((), ())),
                           preferred_element_type=jnp.float32)


def _tc_combine(isd6, x0p, x0n, e2, acc1, acc2, W0, b0, W1, b1, Wa, ba, Wq):
    full = jax.ShapeDtypeStruct((_NP, _D), jnp.float32)
    wspec = pl.BlockSpec((_D, _D), lambda i: (0, 0))
    bspec = pl.BlockSpec((1, _D), lambda i: (0, 0))
    qspec = pl.BlockSpec((1, _D), lambda i: (0, 0))

    def body(isd_ref, xp_ref, xn_ref, e2_ref,
             a1pA, a1pB, a1nA, a1nB, a1mA, a1mB,
             a2pA, a2pB, a2nA, a2nB, a2mA, a2mB,
             W0r, b0r, W1r, b1r, War, bar, Wqr,
             ozc, osn, osn1):
        isd = isd_ref[...]

        def comb(x0, aA1, aB1, aA2, aB2, s):
            accsum = jnp.concatenate(
                [aA1[...] + aA2[...], aB1[...] + aB2[...]], axis=1)
            return (x0 + isd[s][:, None] * accsum) * (1.0 / 3.0)

        sp = comb(xp_ref[...], a1pA, a1pB, a2pA, a2pB, 1)
        sn = comb(xn_ref[...], a1nA, a1nB, a2nA, a2nB, 3)
        sn1 = comb(xn_ref[...], a1mA, a1mB, a2mA, a2mB, 5)

        h = jnp.maximum(_dot_t(e2_ref[...], W0r) + b0r[...], 0.0)
        zng = jnp.maximum(_dot_t(h, W1r) + b1r[...], 0.0)

        wp = _dot_t(jnp.tanh(_dot_t(sp, War) + bar[...]), Wqr)
        wn = _dot_t(jnp.tanh(_dot_t(zng, War) + bar[...]), Wqr)
        m = jnp.maximum(wp, wn)
        e0 = jnp.exp(wp - m)
        e1 = jnp.exp(wn - m)
        a0 = e0 / (e0 + e1)
        ozc[...] = a0 * sp + (1.0 - a0) * zng
        osn[...] = sn
        osn1[...] = sn1

    return pl.pallas_call(
        body,
        grid=(_GRID,),
        in_specs=([_isd_spec(), _full_spec(), _full_spec(), _full_spec()]
                  + [_half_spec()] * 12
                  + [wspec, bspec, wspec, bspec, wspec, bspec, qspec]),
        out_specs=[_full_spec()] * 3,
        out_shape=[full] * 3,
    )(isd6, x0p, x0n, e2, *acc1, *acc2, W0, b0, W1, b1, Wa, ba, Wq)


def _log_sigmoid(x):
    return jnp.minimum(x, 0.0) - jnp.log1p(jnp.exp(-jnp.abs(x)))


def _rnorm(a):
    return a * lax.rsqrt(jnp.maximum(jnp.sum(a * a, axis=1, keepdims=True),
                                     1e-24))


def _tc_loss(u1, v1, u2, v2, u3, v3, n1, n2, wsg):
    RB2 = 512
    NB2 = _B // RB2
    bspec = pl.BlockSpec((RB2, _D), lambda i: (i, 0))
    nspec = pl.BlockSpec((RB2, _K, _D), lambda i: (i, 0, 0))
    wspec = pl.BlockSpec((RB2, 1), lambda i: (i, 0))
    fullspec = pl.BlockSpec((_B, _D), lambda i: (0, 0))

    def body(u1r, v1r, u2r, v2r, u3r, v3r, n1r, n2r, wr, u3f, v3f,
             out, diag_u, diag_v, accs):
        i = pl.program_id(0)

        @pl.when(i == 0)
        def _():
            for k in range(6):
                accs[k] = 0.0

        u1b = u1r[...]
        v1b = v1r[...]
        u2b = u2r[...]
        v2b = v2r[...]
        n1b = n1r[...]
        n2b = n2r[...]
        sgn = jnp.sign(wr[...])

        pos1 = jnp.sum(u1b * v1b, axis=1, keepdims=True)
        neg1 = jnp.sum(u1b[:, None, :] * n1b, axis=2)
        sb1 = jnp.sum(_log_sigmoid((-sgn + 2.0) * pos1 - neg1))
        r1 = jnp.sum(u1b * u1b) + jnp.sum(v1b * v1b) + jnp.sum(n1b * n1b)

        pos2 = jnp.sum(u2b * v2b, axis=1, keepdims=True)
        neg2 = jnp.sum(u2b[:, None, :] * n2b, axis=2)
        sb2 = jnp.sum(_log_sigmoid(neg2 - (sgn + 2.0) * pos2))
        r2 = jnp.sum(u2b * u2b) + jnp.sum(v2b * v2b) + jnp.sum(n2b * n2b)

        u2n = _rnorm(u2b)
        v2n = _rnorm(v2b)
        u3n_blk = _rnorm(u3r[...])
        v3n_blk = _rnorm(v3r[...])
        u3n_all = _rnorm(u3f[...])
        v3n_all = _rnorm(v3f[...])
        fu = jnp.exp(lax.dot_general(u2n, u3n_all, (((1,), (1,)), ((), ())),
                                     preferred_element_type=jnp.float32)
                     / _TAU)
        fv = jnp.exp(lax.dot_general(v2n, v3n_all, (((1,), (1,)), ((), ())),
                                     preferred_element_type=jnp.float32)
                     / _TAU)
        du = jnp.exp(jnp.sum(u2n * u3n_blk, axis=1, keepdims=True) / _TAU)
        dv = jnp.exp(jnp.sum(v2n * v3n_blk, axis=1, keepdims=True) / _TAU)
        diag_u[pl.ds(i * RB2, RB2), :] = du
        diag_v[pl.ds(i * RB2, RB2), :] = dv

        accs[0] = accs[0] + sb1
        accs[1] = accs[1] + r1
        accs[2] = accs[2] + sb2
        accs[3] = accs[3] + r2
        accs[4] = accs[4] + jnp.sum(fu)
        accs[5] = accs[5] + jnp.sum(fv)

        @pl.when(i == NB2 - 1)
        def _():
            du_all = diag_u[...]
            dv_all = diag_v[...]
            pos = du_all + dv_all
            neg = (accs[4] + accs[5]) - du_all - dv_all
            cl = -jnp.log(pos / (pos + neg))
            cl_mean = jnp.sum(cl) / float(_B)
            loss = (-accs[0] + _REG * accs[1]
                    + (-accs[2] / float(_B) + _REG * accs[3])
                    + cl_mean)
            out[0, 0] = loss

    return pl.pallas_call(
        body,
        grid=(NB2,),
        in_specs=[bspec] * 6 + [nspec, nspec, wspec, fullspec, fullspec],
        out_specs=pl.BlockSpec(memory_space=pltpu.SMEM),
        out_shape=jax.ShapeDtypeStruct((1, 1), jnp.float32),
        scratch_shapes=[
            pltpu.VMEM((_B, 1), jnp.float32),
            pltpu.VMEM((_B, 1), jnp.float32),
            pltpu.SMEM((8,), jnp.float32),
        ],
    )(u1, v1, u2, v2, u3, v3, n1, n2, wsg, u3, v3)


def kernel(u, v, w, n, edge_index_p, edge_index_n, edge_index_n1,
           E_pos, E_neg, E_item, E_item_n, E2,
           W0, b0, W1, b1, Wa, ba, Wq):
    f32 = jnp.float32
    i32 = jnp.int32
    npad = _NP - _N
    pe = _NEP - _NE
    fill = (_N + (jnp.arange(pe, dtype=i32) % npad)).astype(i32)

    def pad_edges(ei):
        src = jnp.concatenate([ei[0].astype(i32), fill])
        dst = jnp.concatenate([ei[1].astype(i32), fill])
        return (src.reshape(_NEP // _CH, _CH), dst.reshape(_NEP // _CH, _CH))

    sp_, dp_ = pad_edges(edge_index_p)
    sn_, dn_ = pad_edges(edge_index_n)
    sm_, dm_ = pad_edges(edge_index_n1)
    idx6 = jnp.stack([sp_, dp_, sn_, dn_, sm_, dm_])

    dpart = _sc_degrees(idx6).reshape(6, _NW, _NP)

    zrows = jnp.zeros((npad, _D), f32)
    x0p = jnp.concatenate([E_pos.astype(f32), E_item.astype(f32), zrows])
    x0n = jnp.concatenate([E_neg.astype(f32), E_item_n.astype(f32), zrows])

    prep = _tc_prep(dpart, x0p, x0n)
    y0 = prep[:6]
    isd6 = prep[6]
    srcs = jnp.stack([sp_, sn_, sm_])
    dsts = jnp.stack([dp_, dn_, dm_])
    acc1 = _sc_gconv(y0, srcs, dsts)
    y1 = _tc_inter(isd6, acc1)
    acc2 = _sc_gconv(y1, srcs, dsts)

    e2p = jnp.concatenate([E2.astype(f32), zrows])
    zc, snt, sn1t = _tc_combine(isd6, x0p, x0n, e2p, acc1, acc2,
                                W0, b0.reshape(1, _D), W1, b1.reshape(1, _D),
                                Wa, ba.reshape(1, _D), Wq)

    u2d = u.astype(i32).reshape(_NW, _CH)
    v2d = v.astype(i32).reshape(_NW, _CH)
    n2d = n.astype(i32).reshape((_B * _K) // _CH, _CH)
    g = _sc_batch_gather(zc, snt, sn1t, u2d, v2d, n2d)
    u1g, v1g, n1g, u2g, v2g, n2g, u3g, v3g = g

    loss = _tc_loss(u1g, v1g, u2g, v2g, u3g, v3g,
                    n1g.reshape(_B, _K, _D), n2g.reshape(_B, _K, _D),
                    w.astype(f32).reshape(_B, 1))
    return loss[0, 0]


# speedup vs baseline: 22.7014x; 1.1389x over previous
import functools

import jax
import jax.numpy as jnp
from jax import lax
from jax.experimental import pallas as pl
from jax.experimental.pallas import tpu as pltpu
from jax.experimental.pallas import tpu_sc as plsc

_M = 30000
_NI = 20000
_N = 50000
_NP = 51200
_D = 64
_HALF = 32
_NE = 800000
_NEP = 819200
_B = 4096
_K = 10
_REG = 1e-4
_TAU = 0.8

_NW = 32
_CH = 128
_NCHUNK = (_NEP // _NW) // _CH
_RPT = _NP // 16
_ZR = 100
_CR = _NP // _CH
_MCH = 80

_RB = 512
_GRID = _NP // _RB


def _mesh():
    return plsc.VectorSubcoreMesh(core_axis_name="c", subcore_axis_name="s")


def _sc_params():
    return pltpu.CompilerParams(needs_layout_passes=False,
                                use_tc_tiling_on_sc=False)


_IDXROWS = 40


def _sc_degrees(idx6):
    @functools.partial(
        pl.kernel,
        out_type=jax.ShapeDtypeStruct((6 * _NW * _NP,), jnp.float32),
        mesh=_mesh(),
        compiler_params=_sc_params(),
        scratch_types=[
            pltpu.VMEM((_NP,), jnp.float32),
            pltpu.VMEM((_IDXROWS, _CH), jnp.int32),
        ],
    )
    def deg_kernel(idx_hbm, out_hbm, counts_v, idxbuf_v):
        c = lax.axis_index("c")
        sid = lax.axis_index("s")
        wid = sid * 2 + c
        zero16 = jnp.zeros((16,), jnp.float32)
        ones16 = jnp.ones((16,), jnp.float32)

        for s in range(6):
            def zb(i, carry):
                counts_v[pl.ds(i * 16, 16)] = zero16
                return carry
            lax.fori_loop(0, _NP // 16, zb, None, unroll=4)

            for blk in range(_NCHUNK // _IDXROWS):
                pltpu.sync_copy(
                    idx_hbm.at[s, pl.ds(wid * _NCHUNK + blk * _IDXROWS,
                                        _IDXROWS)],
                    idxbuf_v)

                def chunk_body(j, carry):
                    for g in range(_CH // 16):
                        iv = idxbuf_v[j, pl.ds(g * 16, 16)]
                        plsc.addupdate_scatter(counts_v, [iv], ones16)
                    return carry

                lax.fori_loop(0, _IDXROWS, chunk_body, None)

            pltpu.sync_copy(counts_v,
                            out_hbm.at[pl.ds((s * _NW + wid) * _NP, _NP)])

    return deg_kernel(idx6)


def _edge_loop(c, wid, g, tA, tB, src_hbm, dst_hbm, srcbuf, dstbuf,
               rows0, rows1, gsem0, gsem1, ssem0, ssem1, acc):
    def start_gather(j, buf, sem):
        @pl.when(c == 0)
        def _():
            pltpu.async_copy(tA.at[srcbuf.at[j]], buf, sem)

        @pl.when(c == 1)
        def _():
            pltpu.async_copy(tB.at[srcbuf.at[j]], buf, sem)

    def wait_gather(buf, sem):
        pltpu.make_async_copy(tA.at[srcbuf.at[0]], buf, sem).wait()

    def start_scatter(j, buf, sem):
        pltpu.async_copy(buf, acc.at[dstbuf.at[j]], sem, add=True)

    def wait_scatter(buf, sem):
        pltpu.make_async_copy(buf, acc.at[dstbuf.at[0]], sem).wait()

    for blk in range(_NCHUNK // _IDXROWS):
        base = wid * _NCHUNK + blk * _IDXROWS
        pltpu.sync_copy(src_hbm.at[g, pl.ds(base, _IDXROWS)], srcbuf)
        pltpu.sync_copy(dst_hbm.at[g, pl.ds(base, _IDXROWS)], dstbuf)

        start_gather(0, rows0, gsem0)

        def pair_body(t, carry):
            wait_gather(rows0, gsem0)
            start_gather(2 * t + 1, rows1, gsem1)
            start_scatter(2 * t, rows0, ssem0)
            wait_gather(rows1, gsem1)
            wait_scatter(rows0, ssem0)

            @pl.when(t < _IDXROWS // 2 - 1)
            def _():
                start_gather(2 * t + 2, rows0, gsem0)

            start_scatter(2 * t + 1, rows1, ssem1)
            wait_scatter(rows1, ssem1)
            return carry

        lax.fori_loop(0, _IDXROWS // 2, pair_body, None)


def _sc_gconv1(tabs, srcs, dsts, wsd3):
    half = jax.ShapeDtypeStruct((_NP, _HALF), jnp.float32)

    @functools.partial(
        pl.kernel,
        out_type=[half] * 12,
        mesh=_mesh(),
        compiler_params=_sc_params(),
        scratch_types=[
            pltpu.VMEM((_IDXROWS, _CH), jnp.int32),
            pltpu.VMEM((_IDXROWS, _CH), jnp.int32),
            pltpu.VMEM((_CH, _HALF), jnp.float32),
            pltpu.VMEM((_CH, _HALF), jnp.float32),
            pltpu.VMEM((_ZR, _HALF), jnp.float32),
            pltpu.VMEM((_RPT,), jnp.float32),
            pltpu.SemaphoreType.DMA,
            pltpu.SemaphoreType.DMA,
            pltpu.SemaphoreType.DMA,
            pltpu.SemaphoreType.DMA,
            pltpu.VMEM_SHARED((_NP, _HALF), jnp.float32),
        ],
    )
    def gconv1_kernel(tA0, tB0, tA1, tB1, tA2, tB2, src_hbm, dst_hbm, wsd_hbm,
                      oA0, oB0, oA1, oB1, oA2, oB2,
                      zA0, zB0, zA1, zB1, zA2, zB2,
                      srcbuf, dstbuf, rows0, rows1, zbuf, wsdbuf,
                      gsem0, gsem1, ssem0, ssem1, acc):
        c = lax.axis_index("c")
        sid = lax.axis_index("s")
        wid = sid * 2 + c
        zero16 = jnp.zeros((16,), jnp.float32)

        def zb(i, carry):
            zbuf[i, pl.ds(0, 16)] = zero16
            zbuf[i, pl.ds(16, 16)] = zero16
            return carry
        lax.fori_loop(0, _ZR, zb, None, unroll=4)

        groups = ((tA0, oA0, zA0, tB0, oB0, zB0),
                  (tA1, oA1, zA1, tB1, oB1, zB1),
                  (tA2, oA2, zA2, tB2, oB2, zB2))
        for g in range(3):
            tA, oA, zA, tB, oB, zB = groups[g]
            for r in range(_RPT // _ZR):
                pltpu.sync_copy(zbuf, acc.at[pl.ds(sid * _RPT + r * _ZR, _ZR)])
            plsc.subcore_barrier()

            _edge_loop(c, wid, g, tA, tB, src_hbm, dst_hbm, srcbuf, dstbuf,
                       rows0, rows1, gsem0, gsem1, ssem0, ssem1, acc)
            plsc.subcore_barrier()

            @pl.when(c == 0)
            def _():
                pltpu.sync_copy(acc.at[pl.ds(sid * _RPT, _RPT)],
                                oA.at[pl.ds(sid * _RPT, _RPT)])

            @pl.when(c == 1)
            def _():
                pltpu.sync_copy(acc.at[pl.ds(sid * _RPT, _RPT)],
                                oB.at[pl.ds(sid * _RPT, _RPT)])

            pltpu.sync_copy(wsd_hbm.at[g, pl.ds(sid * _RPT, _RPT)], wsdbuf)

            def chunk_body(ck, carry):
                row0 = sid * _RPT + ck * _CH
                pltpu.sync_copy(acc.at[pl.ds(row0, _CH)], rows0)

                def scale_blk(b, carry2):
                    sv = wsdbuf[pl.ds(ck * _CH + b * 16, 16)]
                    for i in range(16):
                        s = sv[i]
                        r = b * 16 + i
                        rows0[r, pl.ds(0, 16)] = rows0[r, pl.ds(0, 16)] * s
                        rows0[r, pl.ds(16, 16)] = rows0[r, pl.ds(16, 16)] * s
                    return carry2

                lax.fori_loop(0, _CH // 16, scale_blk, None)

                @pl.when(c == 0)
                def _():
                    pltpu.sync_copy(rows0, zA.at[pl.ds(row0, _CH)])

                @pl.when(c == 1)
                def _():
                    pltpu.sync_copy(rows0, zB.at[pl.ds(row0, _CH)])

                return carry

            lax.fori_loop(0, _RPT // _CH, chunk_body, None)

    return gconv1_kernel(*tabs, srcs, dsts, wsd3)


def _sc_gconv2(tabs, srcs, dsts, accin):
    half = jax.ShapeDtypeStruct((_NP, _HALF), jnp.float32)

    @functools.partial(
        pl.kernel,
        out_type=[half] * 6,
        mesh=_mesh(),
        compiler_params=_sc_params(),
        scratch_types=[
            pltpu.VMEM((_IDXROWS, _CH), jnp.int32),
            pltpu.VMEM((_IDXROWS, _CH), jnp.int32),
            pltpu.VMEM((_CH, _HALF), jnp.float32),
            pltpu.VMEM((_CH, _HALF), jnp.float32),
            pltpu.SemaphoreType.DMA,
            pltpu.SemaphoreType.DMA,
            pltpu.SemaphoreType.DMA,
            pltpu.SemaphoreType.DMA,
            pltpu.VMEM_SHARED((_NP, _HALF), jnp.float32),
        ],
    )
    def gconv2_kernel(tA0, tB0, tA1, tB1, tA2, tB2, src_hbm, dst_hbm,
                      pA0, pB0, pA1, pB1, pA2, pB2,
                      oA0, oB0, oA1, oB1, oA2, oB2,
                      srcbuf, dstbuf, rows0, rows1,
                      gsem0, gsem1, ssem0, ssem1, acc):
        c = lax.axis_index("c")
        sid = lax.axis_index("s")
        wid = sid * 2 + c

        groups = ((tA0, pA0, oA0, tB0, pB0, oB0),
                  (tA1, pA1, oA1, tB1, pB1, oB1),
                  (tA2, pA2, oA2, tB2, pB2, oB2))
        for g in range(3):
            tA, pA, oA, tB, pB, oB = groups[g]

            @pl.when(c == 0)
            def _():
                pltpu.sync_copy(pA.at[pl.ds(sid * _RPT, _RPT)],
                                acc.at[pl.ds(sid * _RPT, _RPT)])

            @pl.when(c == 1)
            def _():
                pltpu.sync_copy(pB.at[pl.ds(sid * _RPT, _RPT)],
                                acc.at[pl.ds(sid * _RPT, _RPT)])

            plsc.subcore_barrier()

            _edge_loop(c, wid, g, tA, tB, src_hbm, dst_hbm, srcbuf, dstbuf,
                       rows0, rows1, gsem0, gsem1, ssem0, ssem1, acc)
            plsc.subcore_barrier()

            @pl.when(c == 0)
            def _():
                pltpu.sync_copy(acc.at[pl.ds(sid * _RPT, _RPT)],
                                oA.at[pl.ds(sid * _RPT, _RPT)])

            @pl.when(c == 1)
            def _():
                pltpu.sync_copy(acc.at[pl.ds(sid * _RPT, _RPT)],
                                oB.at[pl.ds(sid * _RPT, _RPT)])

    return gconv2_kernel(*tabs, srcs, dsts, *accin)


def _sc_batch_gather(zc, sn, sn1, u2d, v2d, n2d):
    full = jax.ShapeDtypeStruct((_B, _D), jnp.float32)
    nfull = jax.ShapeDtypeStruct((_B * _K, _D), jnp.float32)

    @functools.partial(
        pl.kernel,
        out_type=[full, full, nfull, full, full, nfull, full, full],
        mesh=_mesh(),
        compiler_params=_sc_params(),
        scratch_types=[
            pltpu.VMEM((_NW, _CH), jnp.int32),
            pltpu.VMEM((_NW, _CH), jnp.int32),
            pltpu.VMEM((_NW * _K, _CH), jnp.int32),
            pltpu.VMEM((_CH, _D), jnp.float32),
            pltpu.SemaphoreType.DMA,
        ],
    )
    def gather_kernel(zc_hbm, sn_hbm, sn1_hbm, u_hbm, v_hbm, n_hbm,
                      ou1, ov1, on1, ou2, ov2, on2, ou3, ov3,
                      ubuf, vbuf, nbuf, rows_v, sem):
        c = lax.axis_index("c")
        sid = lax.axis_index("s")
        wid = sid * 2 + c

        pltpu.sync_copy(u_hbm, ubuf)
        pltpu.sync_copy(v_hbm, vbuf)
        pltpu.sync_copy(n_hbm, nbuf)

        for tab, ibuf, out in ((zc_hbm, ubuf, ou1), (zc_hbm, vbuf, ov1),
                               (sn_hbm, ubuf, ou2), (sn_hbm, vbuf, ov2),
                               (sn1_hbm, ubuf, ou3), (sn1_hbm, vbuf, ov3)):
            pltpu.async_copy(tab.at[ibuf.at[wid]], rows_v, sem).wait()
            pltpu.sync_copy(rows_v, out.at[pl.ds(wid * _CH, _CH)])

        for tab, out in ((zc_hbm, on1), (sn_hbm, on2)):
            for r in range(_K):
                pltpu.async_copy(tab.at[nbuf.at[wid * _K + r]],
                                 rows_v, sem).wait()
                pltpu.sync_copy(rows_v,
                                out.at[pl.ds(wid * _K * _CH + r * _CH, _CH)])

    return gather_kernel(zc, sn, sn1, u2d, v2d, n2d)


def _isd_spec():
    return pl.BlockSpec((6, _RB), lambda i: (0, i))


def _half_spec():
    return pl.BlockSpec((_RB, _HALF), lambda i: (i, 0))


def _full_spec():
    return pl.BlockSpec((_RB, _D), lambda i: (i, 0))


def _tc_prep(dpart, x0p, x0n):
    half = jax.ShapeDtypeStruct((_NP, _HALF), jnp.float32)
    isd6 = jax.ShapeDtypeStruct((6, _NP), jnp.float32)
    wsd3 = jax.ShapeDtypeStruct((3, _NP), jnp.float32)

    def body(dp_ref, xp_ref, xn_ref, opA, opB, onA, onB, om1A, om1B, oisd,
             owsd):
        deg = jnp.sum(dp_ref[...], axis=1)
        isd = lax.rsqrt(jnp.maximum(deg, 1.0))
        oisd[...] = isd
        owsd[...] = jnp.stack([isd[0] * isd[1], isd[2] * isd[3],
                               isd[4] * isd[5]])
        xp = xp_ref[...]
        xn = xn_ref[...]
        yp = xp * isd[0][:, None]
        opA[...] = yp[:, :_HALF]
        opB[...] = yp[:, _HALF:]
        yn = xn * isd[2][:, None]
        onA[...] = yn[:, :_HALF]
        onB[...] = yn[:, _HALF:]
        ym = xn * isd[4][:, None]
        om1A[...] = ym[:, :_HALF]
        om1B[...] = ym[:, _HALF:]

    return pl.pallas_call(
        body,
        grid=(_GRID,),
        in_specs=[pl.BlockSpec((6, _NW, _RB), lambda i: (0, 0, i)),
                  _full_spec(), _full_spec()],
        out_specs=[_half_spec()] * 6 + [_isd_spec(),
                                        pl.BlockSpec((3, _RB),
                                                     lambda i: (0, i))],
        out_shape=[half] * 6 + [isd6, wsd3],
    )(dpart, x0p, x0n)


def _dot_t(x, wref):
    return lax.dot_general(x, wref[...], (((1,), (1,)), ((), ())),
                           preferred_element_type=jnp.float32)


def _tc_combine(isd6, x0p, x0n, e2, accsum, W0, b0, W1, b1, Wa, ba, Wq):
    full = jax.ShapeDtypeStruct((_NP, _D), jnp.float32)
    wspec = pl.BlockSpec((_D, _D), lambda i: (0, 0))
    bspec = pl.BlockSpec((1, _D), lambda i: (0, 0))
    qspec = pl.BlockSpec((1, _D), lambda i: (0, 0))

    def body(isd_ref, xp_ref, xn_ref, e2_ref,
             apA, apB, anA, anB, amA, amB,
             W0r, b0r, W1r, b1r, War, bar, Wqr,
             ozc, osn, osn1):
        isd = isd_ref[...]

        def comb(x0, aA, aB, s):
            accsum_b = jnp.concatenate([aA[...], aB[...]], axis=1)
            return (x0 + isd[s][:, None] * accsum_b) * (1.0 / 3.0)

        sp = comb(xp_ref[...], apA, apB, 1)
        sn = comb(xn_ref[...], anA, anB, 3)
        sn1 = comb(xn_ref[...], amA, amB, 5)

        h = jnp.maximum(_dot_t(e2_ref[...], W0r) + b0r[...], 0.0)
        zng = jnp.maximum(_dot_t(h, W1r) + b1r[...], 0.0)

        wp = _dot_t(jnp.tanh(_dot_t(sp, War) + bar[...]), Wqr)
        wn = _dot_t(jnp.tanh(_dot_t(zng, War) + bar[...]), Wqr)
        m = jnp.maximum(wp, wn)
        e0 = jnp.exp(wp - m)
        e1 = jnp.exp(wn - m)
        a0 = e0 / (e0 + e1)
        ozc[...] = a0 * sp + (1.0 - a0) * zng
        osn[...] = sn
        osn1[...] = sn1

    return pl.pallas_call(
        body,
        grid=(_GRID,),
        in_specs=([_isd_spec(), _full_spec(), _full_spec(), _full_spec()]
                  + [_half_spec()] * 6
                  + [wspec, bspec, wspec, bspec, wspec, bspec, qspec]),
        out_specs=[_full_spec()] * 3,
        out_shape=[full] * 3,
    )(isd6, x0p, x0n, e2, *accsum, W0, b0, W1, b1, Wa, ba, Wq)


def _log_sigmoid(x):
    return jnp.minimum(x, 0.0) - jnp.log1p(jnp.exp(-jnp.abs(x)))


def _rnorm(a):
    return a * lax.rsqrt(jnp.maximum(jnp.sum(a * a, axis=1, keepdims=True),
                                     1e-24))


def _tc_loss(u1, v1, u2, v2, u3, v3, n1, n2, wsg):
    RB2 = 512
    NB2 = _B // RB2
    bspec = pl.BlockSpec((RB2, _D), lambda i: (i, 0))
    nspec = pl.BlockSpec((RB2, _K, _D), lambda i: (i, 0, 0))
    wspec = pl.BlockSpec((RB2, 1), lambda i: (i, 0))
    fullspec = pl.BlockSpec((_B, _D), lambda i: (0, 0))

    def body(u1r, v1r, u2r, v2r, u3r, v3r, n1r, n2r, wr, u3f, v3f,
             out, diag_u, diag_v, accs):
        i = pl.program_id(0)

        @pl.when(i == 0)
        def _():
            for k in range(6):
                accs[k] = 0.0

        u1b = u1r[...]
        v1b = v1r[...]
        u2b = u2r[...]
        v2b = v2r[...]
        n1b = n1r[...]
        n2b = n2r[...]
        sgn = jnp.sign(wr[...])

        pos1 = jnp.sum(u1b * v1b, axis=1, keepdims=True)
        neg1 = jnp.sum(u1b[:, None, :] * n1b, axis=2)
        sb1 = jnp.sum(_log_sigmoid((-sgn + 2.0) * pos1 - neg1))
        r1 = jnp.sum(u1b * u1b) + jnp.sum(v1b * v1b) + jnp.sum(n1b * n1b)

        pos2 = jnp.sum(u2b * v2b, axis=1, keepdims=True)
        neg2 = jnp.sum(u2b[:, None, :] * n2b, axis=2)
        sb2 = jnp.sum(_log_sigmoid(neg2 - (sgn + 2.0) * pos2))
        r2 = jnp.sum(u2b * u2b) + jnp.sum(v2b * v2b) + jnp.sum(n2b * n2b)

        u2n = _rnorm(u2b)
        v2n = _rnorm(v2b)
        u3n_blk = _rnorm(u3r[...])
        v3n_blk = _rnorm(v3r[...])
        u3n_all = _rnorm(u3f[...])
        v3n_all = _rnorm(v3f[...])
        fu = jnp.exp(lax.dot_general(u2n, u3n_all, (((1,), (1,)), ((), ())),
                                     preferred_element_type=jnp.float32)
                     / _TAU)
        fv = jnp.exp(lax.dot_general(v2n, v3n_all, (((1,), (1,)), ((), ())),
                                     preferred_element_type=jnp.float32)
                     / _TAU)
        du = jnp.exp(jnp.sum(u2n * u3n_blk, axis=1, keepdims=True) / _TAU)
        dv = jnp.exp(jnp.sum(v2n * v3n_blk, axis=1, keepdims=True) / _TAU)
        diag_u[pl.ds(i * RB2, RB2), :] = du
        diag_v[pl.ds(i * RB2, RB2), :] = dv

        accs[0] = accs[0] + sb1
        accs[1] = accs[1] + r1
        accs[2] = accs[2] + sb2
        accs[3] = accs[3] + r2
        accs[4] = accs[4] + jnp.sum(fu)
        accs[5] = accs[5] + jnp.sum(fv)

        @pl.when(i == NB2 - 1)
        def _():
            du_all = diag_u[...]
            dv_all = diag_v[...]
            pos = du_all + dv_all
            neg = (accs[4] + accs[5]) - du_all - dv_all
            cl = -jnp.log(pos / (pos + neg))
            cl_mean = jnp.sum(cl) / float(_B)
            loss = (-accs[0] + _REG * accs[1]
                    + (-accs[2] / float(_B) + _REG * accs[3])
                    + cl_mean)
            out[0, 0] = loss

    return pl.pallas_call(
        body,
        grid=(NB2,),
        in_specs=[bspec] * 6 + [nspec, nspec, wspec, fullspec, fullspec],
        out_specs=pl.BlockSpec(memory_space=pltpu.SMEM),
        out_shape=jax.ShapeDtypeStruct((1, 1), jnp.float32),
        scratch_shapes=[
            pltpu.VMEM((_B, 1), jnp.float32),
            pltpu.VMEM((_B, 1), jnp.float32),
            pltpu.SMEM((8,), jnp.float32),
        ],
    )(u1, v1, u2, v2, u3, v3, n1, n2, wsg, u3, v3)


def kernel(u, v, w, n, edge_index_p, edge_index_n, edge_index_n1,
           E_pos, E_neg, E_item, E_item_n, E2,
           W0, b0, W1, b1, Wa, ba, Wq):
    f32 = jnp.float32
    i32 = jnp.int32
    npad = _NP - _N
    pe = _NEP - _NE
    fill = (_N + (jnp.arange(pe, dtype=i32) % npad)).astype(i32)

    def pad_edges(ei):
        src = jnp.concatenate([ei[0].astype(i32), fill])
        dst = jnp.concatenate([ei[1].astype(i32), fill])
        return (src.reshape(_NEP // _CH, _CH), dst.reshape(_NEP // _CH, _CH))

    sp_, dp_ = pad_edges(edge_index_p)
    sn_, dn_ = pad_edges(edge_index_n)
    sm_, dm_ = pad_edges(edge_index_n1)
    idx6 = jnp.stack([sp_, dp_, sn_, dn_, sm_, dm_])

    dpart = _sc_degrees(idx6).reshape(6, _NW, _NP)

    zrows = jnp.zeros((npad, _D), f32)
    x0p = jnp.concatenate([E_pos.astype(f32), E_item.astype(f32), zrows])
    x0n = jnp.concatenate([E_neg.astype(f32), E_item_n.astype(f32), zrows])

    prep = _tc_prep(dpart, x0p, x0n)
    y0 = prep[:6]
    isd6 = prep[6]
    wsd3 = prep[7]
    srcs = jnp.stack([sp_, sn_, sm_])
    dsts = jnp.stack([dp_, dn_, dm_])
    l1 = _sc_gconv1(y0, srcs, dsts, wsd3)
    acc1 = l1[:6]
    z1 = l1[6:]
    accsum = _sc_gconv2(z1, srcs, dsts, acc1)

    e2p = jnp.concatenate([E2.astype(f32), zrows])
    zc, snt, sn1t = _tc_combine(isd6, x0p, x0n, e2p, accsum,
                                W0, b0.reshape(1, _D), W1, b1.reshape(1, _D),
                                Wa, ba.reshape(1, _D), Wq)

    u2d = u.astype(i32).reshape(_NW, _CH)
    v2d = v.astype(i32).reshape(_NW, _CH)
    n2d = n.astype(i32).reshape((_B * _K) // _CH, _CH)
    g = _sc_batch_gather(zc, snt, sn1t, u2d, v2d, n2d)
    u1g, v1g, n1g, u2g, v2g, n2g, u3g, v3g = g

    loss = _tc_loss(u1g, v1g, u2g, v2g, u3g, v3g,
                    n1g.reshape(_B, _K, _D), n2g.reshape(_B, _K, _D),
                    w.astype(f32).reshape(_B, 1))
    return loss[0, 0]


# speedup vs baseline: 28.6740x; 1.2631x over previous
import functools

import jax
import jax.numpy as jnp
from jax import lax
from jax.experimental import pallas as pl
from jax.experimental.pallas import tpu as pltpu
from jax.experimental.pallas import tpu_sc as plsc

_M = 30000
_NI = 20000
_N = 50000
_NP = 51200
_D = 64
_HALF = 32
_NE = 800000
_NEP = 819200
_B = 4096
_K = 10
_REG = 1e-4
_TAU = 0.8

_NW = 32
_CH = 128
_NCHUNK = (_NEP // _NW) // _CH
_RPT = _NP // 16
_ZR = 100
_CR = _NP // _CH
_MCH = 80

_RB = 512
_GRID = _NP // _RB


def _mesh():
    return plsc.VectorSubcoreMesh(core_axis_name="c", subcore_axis_name="s")


def _sc_params():
    return pltpu.CompilerParams(needs_layout_passes=False,
                                use_tc_tiling_on_sc=False)


_IDXROWS = 40
_IR = 20
_NB = 4


def _sc_degrees(idx6):
    @functools.partial(
        pl.kernel,
        out_type=jax.ShapeDtypeStruct((6 * _NW * _NP,), jnp.float32),
        mesh=_mesh(),
        compiler_params=_sc_params(),
        scratch_types=[
            pltpu.VMEM((_NP,), jnp.float32),
            pltpu.VMEM((_IDXROWS, _CH), jnp.int32),
        ],
    )
    def deg_kernel(idx_hbm, out_hbm, counts_v, idxbuf_v):
        c = lax.axis_index("c")
        sid = lax.axis_index("s")
        wid = sid * 2 + c
        zero16 = jnp.zeros((16,), jnp.float32)
        ones16 = jnp.ones((16,), jnp.float32)

        for s in range(6):
            def zb(i, carry):
                counts_v[pl.ds(i * 16, 16)] = zero16
                return carry
            lax.fori_loop(0, _NP // 16, zb, None, unroll=4)

            for blk in range(_NCHUNK // _IDXROWS):
                pltpu.sync_copy(
                    idx_hbm.at[s, pl.ds(wid * _NCHUNK + blk * _IDXROWS,
                                        _IDXROWS)],
                    idxbuf_v)

                def chunk_body(j, carry):
                    for g in range(_CH // 16):
                        iv = idxbuf_v[j, pl.ds(g * 16, 16)]
                        plsc.addupdate_scatter(counts_v, [iv], ones16)
                    return carry

                lax.fori_loop(0, _IDXROWS, chunk_body, None)

            pltpu.sync_copy(counts_v,
                            out_hbm.at[pl.ds((s * _NW + wid) * _NP, _NP)])

    return deg_kernel(idx6)


def _edge_loop(c, wid, g, tA, tB, src_hbm, dst_hbm, srcbuf, dstbuf,
               rows, gsems, ssems, acc):
    nb = len(rows)

    def start_gather(j, buf, sem):
        @pl.when(c == 0)
        def _():
            pltpu.async_copy(tA.at[srcbuf.at[j]], buf, sem)

        @pl.when(c == 1)
        def _():
            pltpu.async_copy(tB.at[srcbuf.at[j]], buf, sem)

    def wait_gather(buf, sem):
        pltpu.make_async_copy(tA.at[srcbuf.at[0]], buf, sem).wait()

    def start_scatter(j, buf, sem):
        pltpu.async_copy(buf, acc.at[dstbuf.at[j]], sem, add=True)

    def wait_scatter(buf, sem):
        pltpu.make_async_copy(buf, acc.at[dstbuf.at[0]], sem).wait()

    nround = _IR // nb
    for blk in range(_NCHUNK // _IR):
        base = wid * _NCHUNK + blk * _IR
        pltpu.sync_copy(src_hbm.at[g, pl.ds(base, _IR)], srcbuf)
        pltpu.sync_copy(dst_hbm.at[g, pl.ds(base, _IR)], dstbuf)

        for k in range(nb):
            start_gather(k, rows[k], gsems[k])

        def round_body(t, carry):
            for k in range(nb):
                wait_gather(rows[k], gsems[k])
                start_scatter(t * nb + k, rows[k], ssems[k])

            @pl.when(t < nround - 1)
            def _():
                for k in range(nb):
                    wait_scatter(rows[k], ssems[k])
                    start_gather((t + 1) * nb + k, rows[k], gsems[k])

            return carry

        lax.fori_loop(0, nround, round_body, None)
        for k in range(nb):
            wait_scatter(rows[k], ssems[k])


def _sc_gconv1(tabs, srcs, dsts, wsd3):
    half = jax.ShapeDtypeStruct((_NP, _HALF), jnp.float32)

    @functools.partial(
        pl.kernel,
        out_type=[half] * 12,
        mesh=_mesh(),
        compiler_params=_sc_params(),
        scratch_types=(
            [pltpu.VMEM((_IR, _CH), jnp.int32),
             pltpu.VMEM((_IR, _CH), jnp.int32)]
            + [pltpu.VMEM((_CH, _HALF), jnp.float32)] * _NB
            + [pltpu.VMEM((_ZR, _HALF), jnp.float32),
               pltpu.VMEM((_RPT,), jnp.float32)]
            + [pltpu.SemaphoreType.DMA] * (2 * _NB)
            + [pltpu.VMEM_SHARED((_NP, _HALF), jnp.float32)]
        ),
    )
    def gconv1_kernel(tA0, tB0, tA1, tB1, tA2, tB2, src_hbm, dst_hbm, wsd_hbm,
                      oA0, oB0, oA1, oB1, oA2, oB2,
                      zA0, zB0, zA1, zB1, zA2, zB2,
                      srcbuf, dstbuf, *rest):
        rows = rest[:_NB]
        zbuf = rest[_NB]
        wsdbuf = rest[_NB + 1]
        gsems = rest[_NB + 2:2 * _NB + 2]
        ssems = rest[2 * _NB + 2:3 * _NB + 2]
        acc = rest[3 * _NB + 2]
        rows0 = rows[0]
        c = lax.axis_index("c")
        sid = lax.axis_index("s")
        wid = sid * 2 + c
        zero16 = jnp.zeros((16,), jnp.float32)

        def zb(i, carry):
            zbuf[i, pl.ds(0, 16)] = zero16
            zbuf[i, pl.ds(16, 16)] = zero16
            return carry
        lax.fori_loop(0, _ZR, zb, None, unroll=4)

        groups = ((tA0, oA0, zA0, tB0, oB0, zB0),
                  (tA1, oA1, zA1, tB1, oB1, zB1),
                  (tA2, oA2, zA2, tB2, oB2, zB2))
        for g in range(3):
            tA, oA, zA, tB, oB, zB = groups[g]
            for r in range(_RPT // _ZR):
                pltpu.sync_copy(zbuf, acc.at[pl.ds(sid * _RPT + r * _ZR, _ZR)])
            plsc.subcore_barrier()

            _edge_loop(c, wid, g, tA, tB, src_hbm, dst_hbm, srcbuf, dstbuf,
                       rows, gsems, ssems, acc)
            plsc.subcore_barrier()

            @pl.when(c == 0)
            def _():
                pltpu.sync_copy(acc.at[pl.ds(sid * _RPT, _RPT)],
                                oA.at[pl.ds(sid * _RPT, _RPT)])

            @pl.when(c == 1)
            def _():
                pltpu.sync_copy(acc.at[pl.ds(sid * _RPT, _RPT)],
                                oB.at[pl.ds(sid * _RPT, _RPT)])

            pltpu.sync_copy(wsd_hbm.at[g, pl.ds(sid * _RPT, _RPT)], wsdbuf)

            def chunk_body(ck, carry):
                row0 = sid * _RPT + ck * _CH
                pltpu.sync_copy(acc.at[pl.ds(row0, _CH)], rows0)

                def scale_blk(b, carry2):
                    sv = wsdbuf[pl.ds(ck * _CH + b * 16, 16)]
                    for i in range(16):
                        s = sv[i]
                        r = b * 16 + i
                        rows0[r, pl.ds(0, 16)] = rows0[r, pl.ds(0, 16)] * s
                        rows0[r, pl.ds(16, 16)] = rows0[r, pl.ds(16, 16)] * s
                    return carry2

                lax.fori_loop(0, _CH // 16, scale_blk, None)

                @pl.when(c == 0)
                def _():
                    pltpu.sync_copy(rows0, zA.at[pl.ds(row0, _CH)])

                @pl.when(c == 1)
                def _():
                    pltpu.sync_copy(rows0, zB.at[pl.ds(row0, _CH)])

                return carry

            lax.fori_loop(0, _RPT // _CH, chunk_body, None)

    return gconv1_kernel(*tabs, srcs, dsts, wsd3)


def _sc_gconv2(tabs, srcs, dsts, accin):
    half = jax.ShapeDtypeStruct((_NP, _HALF), jnp.float32)

    @functools.partial(
        pl.kernel,
        out_type=[half] * 6,
        mesh=_mesh(),
        compiler_params=_sc_params(),
        scratch_types=(
            [pltpu.VMEM((_IR, _CH), jnp.int32),
             pltpu.VMEM((_IR, _CH), jnp.int32)]
            + [pltpu.VMEM((_CH, _HALF), jnp.float32)] * _NB
            + [pltpu.SemaphoreType.DMA] * (2 * _NB)
            + [pltpu.VMEM_SHARED((_NP, _HALF), jnp.float32)]
        ),
    )
    def gconv2_kernel(tA0, tB0, tA1, tB1, tA2, tB2, src_hbm, dst_hbm,
                      pA0, pB0, pA1, pB1, pA2, pB2,
                      oA0, oB0, oA1, oB1, oA2, oB2,
                      srcbuf, dstbuf, *rest):
        rows = rest[:_NB]
        gsems = rest[_NB:2 * _NB]
        ssems = rest[2 * _NB:3 * _NB]
        acc = rest[3 * _NB]
        c = lax.axis_index("c")
        sid = lax.axis_index("s")
        wid = sid * 2 + c

        groups = ((tA0, pA0, oA0, tB0, pB0, oB0),
                  (tA1, pA1, oA1, tB1, pB1, oB1),
                  (tA2, pA2, oA2, tB2, pB2, oB2))
        for g in range(3):
            tA, pA, oA, tB, pB, oB = groups[g]

            @pl.when(c == 0)
            def _():
                pltpu.sync_copy(pA.at[pl.ds(sid * _RPT, _RPT)],
                                acc.at[pl.ds(sid * _RPT, _RPT)])

            @pl.when(c == 1)
            def _():
                pltpu.sync_copy(pB.at[pl.ds(sid * _RPT, _RPT)],
                                acc.at[pl.ds(sid * _RPT, _RPT)])

            plsc.subcore_barrier()

            _edge_loop(c, wid, g, tA, tB, src_hbm, dst_hbm, srcbuf, dstbuf,
                       rows, gsems, ssems, acc)
            plsc.subcore_barrier()

            @pl.when(c == 0)
            def _():
                pltpu.sync_copy(acc.at[pl.ds(sid * _RPT, _RPT)],
                                oA.at[pl.ds(sid * _RPT, _RPT)])

            @pl.when(c == 1)
            def _():
                pltpu.sync_copy(acc.at[pl.ds(sid * _RPT, _RPT)],
                                oB.at[pl.ds(sid * _RPT, _RPT)])

    return gconv2_kernel(*tabs, srcs, dsts, *accin)


def _sc_batch_gather(zc, sn, sn1, u2d, v2d, n2d):
    full = jax.ShapeDtypeStruct((_B, _D), jnp.float32)
    nfull = jax.ShapeDtypeStruct((_B * _K, _D), jnp.float32)

    @functools.partial(
        pl.kernel,
        out_type=[full, full, nfull, full, full, nfull, full, full],
        mesh=_mesh(),
        compiler_params=_sc_params(),
        scratch_types=[
            pltpu.VMEM((_NW, _CH), jnp.int32),
            pltpu.VMEM((_NW, _CH), jnp.int32),
            pltpu.VMEM((_NW * _K, _CH), jnp.int32),
            pltpu.VMEM((_CH, _D), jnp.float32),
            pltpu.SemaphoreType.DMA,
        ],
    )
    def gather_kernel(zc_hbm, sn_hbm, sn1_hbm, u_hbm, v_hbm, n_hbm,
                      ou1, ov1, on1, ou2, ov2, on2, ou3, ov3,
                      ubuf, vbuf, nbuf, rows_v, sem):
        c = lax.axis_index("c")
        sid = lax.axis_index("s")
        wid = sid * 2 + c

        pltpu.sync_copy(u_hbm, ubuf)
        pltpu.sync_copy(v_hbm, vbuf)
        pltpu.sync_copy(n_hbm, nbuf)

        for tab, ibuf, out in ((zc_hbm, ubuf, ou1), (zc_hbm, vbuf, ov1),
                               (sn_hbm, ubuf, ou2), (sn_hbm, vbuf, ov2),
                               (sn1_hbm, ubuf, ou3), (sn1_hbm, vbuf, ov3)):
            pltpu.async_copy(tab.at[ibuf.at[wid]], rows_v, sem).wait()
            pltpu.sync_copy(rows_v, out.at[pl.ds(wid * _CH, _CH)])

        for tab, out in ((zc_hbm, on1), (sn_hbm, on2)):
            for r in range(_K):
                pltpu.async_copy(tab.at[nbuf.at[wid * _K + r]],
                                 rows_v, sem).wait()
                pltpu.sync_copy(rows_v,
                                out.at[pl.ds(wid * _K * _CH + r * _CH, _CH)])

    return gather_kernel(zc, sn, sn1, u2d, v2d, n2d)


def _isd_spec():
    return pl.BlockSpec((6, _RB), lambda i: (0, i))


def _half_spec():
    return pl.BlockSpec((_RB, _HALF), lambda i: (i, 0))


def _full_spec():
    return pl.BlockSpec((_RB, _D), lambda i: (i, 0))


def _tc_prep(dpart, x0p, x0n):
    half = jax.ShapeDtypeStruct((_NP, _HALF), jnp.float32)
    isd6 = jax.ShapeDtypeStruct((6, _NP), jnp.float32)
    wsd3 = jax.ShapeDtypeStruct((3, _NP), jnp.float32)

    def body(dp_ref, xp_ref, xn_ref, opA, opB, onA, onB, om1A, om1B, oisd,
             owsd):
        deg = jnp.sum(dp_ref[...], axis=1)
        isd = lax.rsqrt(jnp.maximum(deg, 1.0))
        oisd[...] = isd
        owsd[...] = jnp.stack([isd[0] * isd[1], isd[2] * isd[3],
                               isd[4] * isd[5]])
        xp = xp_ref[...]
        xn = xn_ref[...]
        yp = xp * isd[0][:, None]
        opA[...] = yp[:, :_HALF]
        opB[...] = yp[:, _HALF:]
        yn = xn * isd[2][:, None]
        onA[...] = yn[:, :_HALF]
        onB[...] = yn[:, _HALF:]
        ym = xn * isd[4][:, None]
        om1A[...] = ym[:, :_HALF]
        om1B[...] = ym[:, _HALF:]

    return pl.pallas_call(
        body,
        grid=(_GRID,),
        in_specs=[pl.BlockSpec((6, _NW, _RB), lambda i: (0, 0, i)),
                  _full_spec(), _full_spec()],
        out_specs=[_half_spec()] * 6 + [_isd_spec(),
                                        pl.BlockSpec((3, _RB),
                                                     lambda i: (0, i))],
        out_shape=[half] * 6 + [isd6, wsd3],
    )(dpart, x0p, x0n)


def _dot_t(x, wref):
    return lax.dot_general(x, wref[...], (((1,), (1,)), ((), ())),
                           preferred_element_type=jnp.float32)


def _tc_combine(isd6, x0p, x0n, e2, accsum, W0, b0, W1, b1, Wa, ba, Wq):
    full = jax.ShapeDtypeStruct((_NP, _D), jnp.float32)
    wspec = pl.BlockSpec((_D, _D), lambda i: (0, 0))
    bspec = pl.BlockSpec((1, _D), lambda i: (0, 0))
    qspec = pl.BlockSpec((1, _D), lambda i: (0, 0))

    def body(isd_ref, xp_ref, xn_ref, e2_ref,
             apA, apB, anA, anB, amA, amB,
             W0r, b0r, W1r, b1r, War, bar, Wqr,
             ozc, osn, osn1):
        isd = isd_ref[...]

        def comb(x0, aA, aB, s):
            accsum_b = jnp.concatenate([aA[...], aB[...]], axis=1)
            return (x0 + isd[s][:, None] * accsum_b) * (1.0 / 3.0)

        sp = comb(xp_ref[...], apA, apB, 1)
        sn = comb(xn_ref[...], anA, anB, 3)
        sn1 = comb(xn_ref[...], amA, amB, 5)

        h = jnp.maximum(_dot_t(e2_ref[...], W0r) + b0r[...], 0.0)
        zng = jnp.maximum(_dot_t(h, W1r) + b1r[...], 0.0)

        wp = _dot_t(jnp.tanh(_dot_t(sp, War) + bar[...]), Wqr)
        wn = _dot_t(jnp.tanh(_dot_t(zng, War) + bar[...]), Wqr)
        m = jnp.maximum(wp, wn)
        e0 = jnp.exp(wp - m)
        e1 = jnp.exp(wn - m)
        a0 = e0 / (e0 + e1)
        ozc[...] = a0 * sp + (1.0 - a0) * zng
        osn[...] = sn
        osn1[...] = sn1

    return pl.pallas_call(
        body,
        grid=(_GRID,),
        in_specs=([_isd_spec(), _full_spec(), _full_spec(), _full_spec()]
                  + [_half_spec()] * 6
                  + [wspec, bspec, wspec, bspec, wspec, bspec, qspec]),
        out_specs=[_full_spec()] * 3,
        out_shape=[full] * 3,
    )(isd6, x0p, x0n, e2, *accsum, W0, b0, W1, b1, Wa, ba, Wq)


def _log_sigmoid(x):
    return jnp.minimum(x, 0.0) - jnp.log1p(jnp.exp(-jnp.abs(x)))


def _rnorm(a):
    return a * lax.rsqrt(jnp.maximum(jnp.sum(a * a, axis=1, keepdims=True),
                                     1e-24))


def _tc_loss(u1, v1, u2, v2, u3, v3, n1, n2, wsg):
    RB2 = 512
    NB2 = _B // RB2
    bspec = pl.BlockSpec((RB2, _D), lambda i: (i, 0))
    nspec = pl.BlockSpec((RB2, _K, _D), lambda i: (i, 0, 0))
    wspec = pl.BlockSpec((RB2, 1), lambda i: (i, 0))
    fullspec = pl.BlockSpec((_B, _D), lambda i: (0, 0))

    def body(u1r, v1r, u2r, v2r, u3r, v3r, n1r, n2r, wr, u3f, v3f,
             out, diag_u, diag_v, accs):
        i = pl.program_id(0)

        @pl.when(i == 0)
        def _():
            for k in range(6):
                accs[k] = 0.0

        u1b = u1r[...]
        v1b = v1r[...]
        u2b = u2r[...]
        v2b = v2r[...]
        n1b = n1r[...]
        n2b = n2r[...]
        sgn = jnp.sign(wr[...])

        pos1 = jnp.sum(u1b * v1b, axis=1, keepdims=True)
        neg1 = jnp.sum(u1b[:, None, :] * n1b, axis=2)
        sb1 = jnp.sum(_log_sigmoid((-sgn + 2.0) * pos1 - neg1))
        r1 = jnp.sum(u1b * u1b) + jnp.sum(v1b * v1b) + jnp.sum(n1b * n1b)

        pos2 = jnp.sum(u2b * v2b, axis=1, keepdims=True)
        neg2 = jnp.sum(u2b[:, None, :] * n2b, axis=2)
        sb2 = jnp.sum(_log_sigmoid(neg2 - (sgn + 2.0) * pos2))
        r2 = jnp.sum(u2b * u2b) + jnp.sum(v2b * v2b) + jnp.sum(n2b * n2b)

        u2n = _rnorm(u2b)
        v2n = _rnorm(v2b)
        u3n_blk = _rnorm(u3r[...])
        v3n_blk = _rnorm(v3r[...])
        u3n_all = _rnorm(u3f[...])
        v3n_all = _rnorm(v3f[...])
        fu = jnp.exp(lax.dot_general(u2n, u3n_all, (((1,), (1,)), ((), ())),
                                     preferred_element_type=jnp.float32)
                     / _TAU)
        fv = jnp.exp(lax.dot_general(v2n, v3n_all, (((1,), (1,)), ((), ())),
                                     preferred_element_type=jnp.float32)
                     / _TAU)
        du = jnp.exp(jnp.sum(u2n * u3n_blk, axis=1, keepdims=True) / _TAU)
        dv = jnp.exp(jnp.sum(v2n * v3n_blk, axis=1, keepdims=True) / _TAU)
        diag_u[pl.ds(i * RB2, RB2), :] = du
        diag_v[pl.ds(i * RB2, RB2), :] = dv

        accs[0] = accs[0] + sb1
        accs[1] = accs[1] + r1
        accs[2] = accs[2] + sb2
        accs[3] = accs[3] + r2
        accs[4] = accs[4] + jnp.sum(fu)
        accs[5] = accs[5] + jnp.sum(fv)

        @pl.when(i == NB2 - 1)
        def _():
            du_all = diag_u[...]
            dv_all = diag_v[...]
            pos = du_all + dv_all
            neg = (accs[4] + accs[5]) - du_all - dv_all
            cl = -jnp.log(pos / (pos + neg))
            cl_mean = jnp.sum(cl) / float(_B)
            loss = (-accs[0] + _REG * accs[1]
                    + (-accs[2] / float(_B) + _REG * accs[3])
                    + cl_mean)
            out[0, 0] = loss

    return pl.pallas_call(
        body,
        grid=(NB2,),
        in_specs=[bspec] * 6 + [nspec, nspec, wspec, fullspec, fullspec],
        out_specs=pl.BlockSpec(memory_space=pltpu.SMEM),
        out_shape=jax.ShapeDtypeStruct((1, 1), jnp.float32),
        scratch_shapes=[
            pltpu.VMEM((_B, 1), jnp.float32),
            pltpu.VMEM((_B, 1), jnp.float32),
            pltpu.SMEM((8,), jnp.float32),
        ],
    )(u1, v1, u2, v2, u3, v3, n1, n2, wsg, u3, v3)


def kernel(u, v, w, n, edge_index_p, edge_index_n, edge_index_n1,
           E_pos, E_neg, E_item, E_item_n, E2,
           W0, b0, W1, b1, Wa, ba, Wq):
    f32 = jnp.float32
    i32 = jnp.int32
    npad = _NP - _N
    pe = _NEP - _NE
    fill = (_N + (jnp.arange(pe, dtype=i32) % npad)).astype(i32)

    def pad_edges(ei):
        src = jnp.concatenate([ei[0].astype(i32), fill])
        dst = jnp.concatenate([ei[1].astype(i32), fill])
        return (src.reshape(_NEP // _CH, _CH), dst.reshape(_NEP // _CH, _CH))

    sp_, dp_ = pad_edges(edge_index_p)
    sn_, dn_ = pad_edges(edge_index_n)
    sm_, dm_ = pad_edges(edge_index_n1)
    idx6 = jnp.stack([sp_, dp_, sn_, dn_, sm_, dm_])

    dpart = _sc_degrees(idx6).reshape(6, _NW, _NP)

    zrows = jnp.zeros((npad, _D), f32)
    x0p = jnp.concatenate([E_pos.astype(f32), E_item.astype(f32), zrows])
    x0n = jnp.concatenate([E_neg.astype(f32), E_item_n.astype(f32), zrows])

    prep = _tc_prep(dpart, x0p, x0n)
    y0 = prep[:6]
    isd6 = prep[6]
    wsd3 = prep[7]
    srcs = jnp.stack([sp_, sn_, sm_])
    dsts = jnp.stack([dp_, dn_, dm_])
    l1 = _sc_gconv1(y0, srcs, dsts, wsd3)
    acc1 = l1[:6]
    z1 = l1[6:]
    accsum = _sc_gconv2(z1, srcs, dsts, acc1)

    e2p = jnp.concatenate([E2.astype(f32), zrows])
    zc, snt, sn1t = _tc_combine(isd6, x0p, x0n, e2p, accsum,
                                W0, b0.reshape(1, _D), W1, b1.reshape(1, _D),
                                Wa, ba.reshape(1, _D), Wq)

    u2d = u.astype(i32).reshape(_NW, _CH)
    v2d = v.astype(i32).reshape(_NW, _CH)
    n2d = n.astype(i32).reshape((_B * _K) // _CH, _CH)
    g = _sc_batch_gather(zc, snt, sn1t, u2d, v2d, n2d)
    u1g, v1g, n1g, u2g, v2g, n2g, u3g, v3g = g

    loss = _tc_loss(u1g, v1g, u2g, v2g, u3g, v3g,
                    n1g.reshape(_B, _K, _D), n2g.reshape(_B, _K, _D),
                    w.astype(f32).reshape(_B, 1))
    return loss[0, 0]


# speedup vs baseline: 29.8035x; 1.0394x over previous
import functools

import jax
import jax.numpy as jnp
from jax import lax
from jax.experimental import pallas as pl
from jax.experimental.pallas import tpu as pltpu
from jax.experimental.pallas import tpu_sc as plsc

_M = 30000
_NI = 20000
_N = 50000
_NP = 51200
_D = 64
_HALF = 32
_NE = 800000
_NEP = 819200
_B = 4096
_K = 10
_REG = 1e-4
_TAU = 0.8

_NW = 32
_CH = 128
_NCHUNK = (_NEP // _NW) // _CH
_RPT = _NP // 16
_ZR = 100
_CR = _NP // _CH
_MCH = 80

_RB = 1024
_GRID = _NP // _RB


def _mesh():
    return plsc.VectorSubcoreMesh(core_axis_name="c", subcore_axis_name="s")


def _sc_params():
    return pltpu.CompilerParams(needs_layout_passes=False,
                                use_tc_tiling_on_sc=False)


_IDXROWS = 40
_IR = 20
_NB = 4


def _sc_degrees(idx6):
    @functools.partial(
        pl.kernel,
        out_type=jax.ShapeDtypeStruct((6 * _NW * _NP,), jnp.float32),
        mesh=_mesh(),
        compiler_params=_sc_params(),
        scratch_types=[
            pltpu.VMEM((_NP,), jnp.float32),
            pltpu.VMEM((_IDXROWS, _CH), jnp.int32),
        ],
    )
    def deg_kernel(idx_hbm, out_hbm, counts_v, idxbuf_v):
        c = lax.axis_index("c")
        sid = lax.axis_index("s")
        wid = sid * 2 + c
        zero16 = jnp.zeros((16,), jnp.float32)
        ones16 = jnp.ones((16,), jnp.float32)

        for s in range(6):
            def zb(i, carry):
                counts_v[pl.ds(i * 16, 16)] = zero16
                return carry
            lax.fori_loop(0, _NP // 16, zb, None, unroll=4)

            for blk in range(_NCHUNK // _IDXROWS):
                pltpu.sync_copy(
                    idx_hbm.at[s, pl.ds(wid * _NCHUNK + blk * _IDXROWS,
                                        _IDXROWS)],
                    idxbuf_v)

                def chunk_body(j, carry):
                    for g in range(_CH // 16):
                        iv = idxbuf_v[j, pl.ds(g * 16, 16)]
                        plsc.addupdate_scatter(counts_v, [iv], ones16)
                    return carry

                lax.fori_loop(0, _IDXROWS, chunk_body, None)

            pltpu.sync_copy(counts_v,
                            out_hbm.at[pl.ds((s * _NW + wid) * _NP, _NP)])

    return deg_kernel(idx6)


def _edge_loop(c, wid, g, tA, tB, src_hbm, dst_hbm, srcbuf, dstbuf,
               rows, gsems, ssems, acc):
    nb = len(rows)

    def start_gather(j, buf, sem):
        @pl.when(c == 0)
        def _():
            pltpu.async_copy(tA.at[srcbuf.at[j]], buf, sem)

        @pl.when(c == 1)
        def _():
            pltpu.async_copy(tB.at[srcbuf.at[j]], buf, sem)

    def wait_gather(buf, sem):
        pltpu.make_async_copy(tA.at[srcbuf.at[0]], buf, sem).wait()

    def start_scatter(j, buf, sem):
        pltpu.async_copy(buf, acc.at[dstbuf.at[j]], sem, add=True)

    def wait_scatter(buf, sem):
        pltpu.make_async_copy(buf, acc.at[dstbuf.at[0]], sem).wait()

    nround = _IR // nb
    for blk in range(_NCHUNK // _IR):
        base = wid * _NCHUNK + blk * _IR
        pltpu.sync_copy(src_hbm.at[g, pl.ds(base, _IR)], srcbuf)
        pltpu.sync_copy(dst_hbm.at[g, pl.ds(base, _IR)], dstbuf)

        for k in range(nb):
            start_gather(k, rows[k], gsems[k])

        def round_body(t, carry):
            for k in range(nb):
                wait_gather(rows[k], gsems[k])
                start_scatter(t * nb + k, rows[k], ssems[k])

            @pl.when(t < nround - 1)
            def _():
                for k in range(nb):
                    wait_scatter(rows[k], ssems[k])
                    start_gather((t + 1) * nb + k, rows[k], gsems[k])

            return carry

        lax.fori_loop(0, nround, round_body, None)
        for k in range(nb):
            wait_scatter(rows[k], ssems[k])


def _sc_gconv1(tabs, srcs, dsts, wsd3):
    half = jax.ShapeDtypeStruct((_NP, _HALF), jnp.float32)

    @functools.partial(
        pl.kernel,
        out_type=[half] * 12,
        mesh=_mesh(),
        compiler_params=_sc_params(),
        scratch_types=(
            [pltpu.VMEM((_IR, _CH), jnp.int32),
             pltpu.VMEM((_IR, _CH), jnp.int32)]
            + [pltpu.VMEM((_CH, _HALF), jnp.float32)] * _NB
            + [pltpu.VMEM((_ZR, _HALF), jnp.float32),
               pltpu.VMEM((_RPT,), jnp.float32)]
            + [pltpu.SemaphoreType.DMA] * (2 * _NB)
            + [pltpu.VMEM_SHARED((_NP, _HALF), jnp.float32)]
        ),
    )
    def gconv1_kernel(tA0, tB0, tA1, tB1, tA2, tB2, src_hbm, dst_hbm, wsd_hbm,
                      oA0, oB0, oA1, oB1, oA2, oB2,
                      zA0, zB0, zA1, zB1, zA2, zB2,
                      srcbuf, dstbuf, *rest):
        rows = rest[:_NB]
        zbuf = rest[_NB]
        wsdbuf = rest[_NB + 1]
        gsems = rest[_NB + 2:2 * _NB + 2]
        ssems = rest[2 * _NB + 2:3 * _NB + 2]
        acc = rest[3 * _NB + 2]
        rows0 = rows[0]
        c = lax.axis_index("c")
        sid = lax.axis_index("s")
        wid = sid * 2 + c
        zero16 = jnp.zeros((16,), jnp.float32)

        def zb(i, carry):
            zbuf[i, pl.ds(0, 16)] = zero16
            zbuf[i, pl.ds(16, 16)] = zero16
            return carry
        lax.fori_loop(0, _ZR, zb, None, unroll=4)

        groups = ((tA0, oA0, zA0, tB0, oB0, zB0),
                  (tA1, oA1, zA1, tB1, oB1, zB1),
                  (tA2, oA2, zA2, tB2, oB2, zB2))
        for g in range(3):
            tA, oA, zA, tB, oB, zB = groups[g]
            for r in range(_RPT // _ZR):
                pltpu.sync_copy(zbuf, acc.at[pl.ds(sid * _RPT + r * _ZR, _ZR)])
            plsc.subcore_barrier()

            _edge_loop(c, wid, g, tA, tB, src_hbm, dst_hbm, srcbuf, dstbuf,
                       rows, gsems, ssems, acc)
            plsc.subcore_barrier()

            @pl.when(c == 0)
            def _():
                pltpu.sync_copy(acc.at[pl.ds(sid * _RPT, _RPT)],
                                oA.at[pl.ds(sid * _RPT, _RPT)])

            @pl.when(c == 1)
            def _():
                pltpu.sync_copy(acc.at[pl.ds(sid * _RPT, _RPT)],
                                oB.at[pl.ds(sid * _RPT, _RPT)])

            pltpu.sync_copy(wsd_hbm.at[g, pl.ds(sid * _RPT, _RPT)], wsdbuf)

            def chunk_body(ck, carry):
                row0 = sid * _RPT + ck * _CH
                pltpu.sync_copy(acc.at[pl.ds(row0, _CH)], rows0)

                def scale_blk(b, carry2):
                    sv = wsdbuf[pl.ds(ck * _CH + b * 16, 16)]
                    for i in range(16):
                        s = sv[i]
                        r = b * 16 + i
                        rows0[r, pl.ds(0, 16)] = rows0[r, pl.ds(0, 16)] * s
                        rows0[r, pl.ds(16, 16)] = rows0[r, pl.ds(16, 16)] * s
                    return carry2

                lax.fori_loop(0, _CH // 16, scale_blk, None)

                @pl.when(c == 0)
                def _():
                    pltpu.sync_copy(rows0, zA.at[pl.ds(row0, _CH)])

                @pl.when(c == 1)
                def _():
                    pltpu.sync_copy(rows0, zB.at[pl.ds(row0, _CH)])

                return carry

            lax.fori_loop(0, _RPT // _CH, chunk_body, None)

    return gconv1_kernel(*tabs, srcs, dsts, wsd3)


def _sc_gconv2(tabs, srcs, dsts, accin):
    half = jax.ShapeDtypeStruct((_NP, _HALF), jnp.float32)

    @functools.partial(
        pl.kernel,
        out_type=[half] * 6,
        mesh=_mesh(),
        compiler_params=_sc_params(),
        scratch_types=(
            [pltpu.VMEM((_IR, _CH), jnp.int32),
             pltpu.VMEM((_IR, _CH), jnp.int32)]
            + [pltpu.VMEM((_CH, _HALF), jnp.float32)] * _NB
            + [pltpu.SemaphoreType.DMA] * (2 * _NB)
            + [pltpu.VMEM_SHARED((_NP, _HALF), jnp.float32)]
        ),
    )
    def gconv2_kernel(tA0, tB0, tA1, tB1, tA2, tB2, src_hbm, dst_hbm,
                      pA0, pB0, pA1, pB1, pA2, pB2,
                      oA0, oB0, oA1, oB1, oA2, oB2,
                      srcbuf, dstbuf, *rest):
        rows = rest[:_NB]
        gsems = rest[_NB:2 * _NB]
        ssems = rest[2 * _NB:3 * _NB]
        acc = rest[3 * _NB]
        c = lax.axis_index("c")
        sid = lax.axis_index("s")
        wid = sid * 2 + c

        groups = ((tA0, pA0, oA0, tB0, pB0, oB0),
                  (tA1, pA1, oA1, tB1, pB1, oB1),
                  (tA2, pA2, oA2, tB2, pB2, oB2))
        for g in range(3):
            tA, pA, oA, tB, pB, oB = groups[g]

            @pl.when(c == 0)
            def _():
                pltpu.sync_copy(pA.at[pl.ds(sid * _RPT, _RPT)],
                                acc.at[pl.ds(sid * _RPT, _RPT)])

            @pl.when(c == 1)
            def _():
                pltpu.sync_copy(pB.at[pl.ds(sid * _RPT, _RPT)],
                                acc.at[pl.ds(sid * _RPT, _RPT)])

            plsc.subcore_barrier()

            _edge_loop(c, wid, g, tA, tB, src_hbm, dst_hbm, srcbuf, dstbuf,
                       rows, gsems, ssems, acc)
            plsc.subcore_barrier()

            @pl.when(c == 0)
            def _():
                pltpu.sync_copy(acc.at[pl.ds(sid * _RPT, _RPT)],
                                oA.at[pl.ds(sid * _RPT, _RPT)])

            @pl.when(c == 1)
            def _():
                pltpu.sync_copy(acc.at[pl.ds(sid * _RPT, _RPT)],
                                oB.at[pl.ds(sid * _RPT, _RPT)])

    return gconv2_kernel(*tabs, srcs, dsts, *accin)


def _sc_batch_gather(zc, sn, sn1, u2d, v2d, n2d):
    full = jax.ShapeDtypeStruct((_B, _D), jnp.float32)
    nfull = jax.ShapeDtypeStruct((_B * _K, _D), jnp.float32)

    @functools.partial(
        pl.kernel,
        out_type=[full, full, nfull, full, full, nfull, full, full],
        mesh=_mesh(),
        compiler_params=_sc_params(),
        scratch_types=[
            pltpu.VMEM((_NW, _CH), jnp.int32),
            pltpu.VMEM((_NW, _CH), jnp.int32),
            pltpu.VMEM((_NW * _K, _CH), jnp.int32),
            pltpu.VMEM((_CH, _D), jnp.float32),
            pltpu.SemaphoreType.DMA,
        ],
    )
    def gather_kernel(zc_hbm, sn_hbm, sn1_hbm, u_hbm, v_hbm, n_hbm,
                      ou1, ov1, on1, ou2, ov2, on2, ou3, ov3,
                      ubuf, vbuf, nbuf, rows_v, sem):
        c = lax.axis_index("c")
        sid = lax.axis_index("s")
        wid = sid * 2 + c

        pltpu.sync_copy(u_hbm, ubuf)
        pltpu.sync_copy(v_hbm, vbuf)
        pltpu.sync_copy(n_hbm, nbuf)

        for tab, ibuf, out in ((zc_hbm, ubuf, ou1), (zc_hbm, vbuf, ov1),
                               (sn_hbm, ubuf, ou2), (sn_hbm, vbuf, ov2),
                               (sn1_hbm, ubuf, ou3), (sn1_hbm, vbuf, ov3)):
            pltpu.async_copy(tab.at[ibuf.at[wid]], rows_v, sem).wait()
            pltpu.sync_copy(rows_v, out.at[pl.ds(wid * _CH, _CH)])

        for tab, out in ((zc_hbm, on1), (sn_hbm, on2)):
            for r in range(_K):
                pltpu.async_copy(tab.at[nbuf.at[wid * _K + r]],
                                 rows_v, sem).wait()
                pltpu.sync_copy(rows_v,
                                out.at[pl.ds(wid * _K * _CH + r * _CH, _CH)])

    return gather_kernel(zc, sn, sn1, u2d, v2d, n2d)


def _isd_spec():
    return pl.BlockSpec((6, _RB), lambda i: (0, i))


def _half_spec():
    return pl.BlockSpec((_RB, _HALF), lambda i: (i, 0))


def _full_spec():
    return pl.BlockSpec((_RB, _D), lambda i: (i, 0))


def _tc_prep(dpart, x0p, x0n):
    half = jax.ShapeDtypeStruct((_NP, _HALF), jnp.float32)
    isd6 = jax.ShapeDtypeStruct((6, _NP), jnp.float32)
    wsd3 = jax.ShapeDtypeStruct((3, _NP), jnp.float32)

    def body(dp_ref, xp_ref, xn_ref, opA, opB, onA, onB, om1A, om1B, oisd,
             owsd):
        deg = jnp.sum(dp_ref[...], axis=1)
        isd = lax.rsqrt(jnp.maximum(deg, 1.0))
        oisd[...] = isd
        owsd[...] = jnp.stack([isd[0] * isd[1], isd[2] * isd[3],
                               isd[4] * isd[5]])
        xp = xp_ref[...]
        xn = xn_ref[...]
        yp = xp * isd[0][:, None]
        opA[...] = yp[:, :_HALF]
        opB[...] = yp[:, _HALF:]
        yn = xn * isd[2][:, None]
        onA[...] = yn[:, :_HALF]
        onB[...] = yn[:, _HALF:]
        ym = xn * isd[4][:, None]
        om1A[...] = ym[:, :_HALF]
        om1B[...] = ym[:, _HALF:]

    return pl.pallas_call(
        body,
        grid=(_GRID,),
        in_specs=[pl.BlockSpec((6, _NW, _RB), lambda i: (0, 0, i)),
                  _full_spec(), _full_spec()],
        out_specs=[_half_spec()] * 6 + [_isd_spec(),
                                        pl.BlockSpec((3, _RB),
                                                     lambda i: (0, i))],
        out_shape=[half] * 6 + [isd6, wsd3],
    )(dpart, x0p, x0n)


def _dot_t(x, wref):
    return lax.dot_general(x, wref[...], (((1,), (1,)), ((), ())),
                           preferred_element_type=jnp.float32)


def _tc_combine(isd6, x0p, x0n, e2, accsum, W0, b0, W1, b1, Wa, ba, Wq):
    full = jax.ShapeDtypeStruct((_NP, _D), jnp.float32)
    wspec = pl.BlockSpec((_D, _D), lambda i: (0, 0))
    bspec = pl.BlockSpec((1, _D), lambda i: (0, 0))
    qspec = pl.BlockSpec((1, _D), lambda i: (0, 0))

    def body(isd_ref, xp_ref, xn_ref, e2_ref,
             apA, apB, anA, anB, amA, amB,
             W0r, b0r, W1r, b1r, War, bar, Wqr,
             ozc, osn, osn1):
        isd = isd_ref[...]

        def comb(x0, aA, aB, s):
            accsum_b = jnp.concatenate([aA[...], aB[...]], axis=1)
            return (x0 + isd[s][:, None] * accsum_b) * (1.0 / 3.0)

        sp = comb(xp_ref[...], apA, apB, 1)
        sn = comb(xn_ref[...], anA, anB, 3)
        sn1 = comb(xn_ref[...], amA, amB, 5)

        h = jnp.maximum(_dot_t(e2_ref[...], W0r) + b0r[...], 0.0)
        zng = jnp.maximum(_dot_t(h, W1r) + b1r[...], 0.0)

        wp = _dot_t(jnp.tanh(_dot_t(sp, War) + bar[...]), Wqr)
        wn = _dot_t(jnp.tanh(_dot_t(zng, War) + bar[...]), Wqr)
        m = jnp.maximum(wp, wn)
        e0 = jnp.exp(wp - m)
        e1 = jnp.exp(wn - m)
        a0 = e0 / (e0 + e1)
        ozc[...] = a0 * sp + (1.0 - a0) * zng
        osn[...] = sn
        osn1[...] = sn1

    return pl.pallas_call(
        body,
        grid=(_GRID,),
        in_specs=([_isd_spec(), _full_spec(), _full_spec(), _full_spec()]
                  + [_half_spec()] * 6
                  + [wspec, bspec, wspec, bspec, wspec, bspec, qspec]),
        out_specs=[_full_spec()] * 3,
        out_shape=[full] * 3,
    )(isd6, x0p, x0n, e2, *accsum, W0, b0, W1, b1, Wa, ba, Wq)


def _log_sigmoid(x):
    return jnp.minimum(x, 0.0) - jnp.log1p(jnp.exp(-jnp.abs(x)))


def _rnorm(a):
    return a * lax.rsqrt(jnp.maximum(jnp.sum(a * a, axis=1, keepdims=True),
                                     1e-24))


def _tc_loss(u1, v1, u2, v2, u3, v3, n1, n2, wsg):
    RB2 = 512
    NB2 = _B // RB2
    bspec = pl.BlockSpec((RB2, _D), lambda i: (i, 0))
    nspec = pl.BlockSpec((RB2, _K, _D), lambda i: (i, 0, 0))
    wspec = pl.BlockSpec((RB2, 1), lambda i: (i, 0))
    fullspec = pl.BlockSpec((_B, _D), lambda i: (0, 0))

    def body(u1r, v1r, u2r, v2r, u3r, v3r, n1r, n2r, wr, u3f, v3f,
             out, diag_u, diag_v, accs):
        i = pl.program_id(0)

        @pl.when(i == 0)
        def _():
            for k in range(6):
                accs[k] = 0.0

        u1b = u1r[...]
        v1b = v1r[...]
        u2b = u2r[...]
        v2b = v2r[...]
        n1b = n1r[...]
        n2b = n2r[...]
        sgn = jnp.sign(wr[...])

        pos1 = jnp.sum(u1b * v1b, axis=1, keepdims=True)
        neg1 = jnp.sum(u1b[:, None, :] * n1b, axis=2)
        sb1 = jnp.sum(_log_sigmoid((-sgn + 2.0) * pos1 - neg1))
        r1 = jnp.sum(u1b * u1b) + jnp.sum(v1b * v1b) + jnp.sum(n1b * n1b)

        pos2 = jnp.sum(u2b * v2b, axis=1, keepdims=True)
        neg2 = jnp.sum(u2b[:, None, :] * n2b, axis=2)
        sb2 = jnp.sum(_log_sigmoid(neg2 - (sgn + 2.0) * pos2))
        r2 = jnp.sum(u2b * u2b) + jnp.sum(v2b * v2b) + jnp.sum(n2b * n2b)

        u2n = _rnorm(u2b)
        v2n = _rnorm(v2b)
        u3n_blk = _rnorm(u3r[...])
        v3n_blk = _rnorm(v3r[...])
        u3n_all = _rnorm(u3f[...])
        v3n_all = _rnorm(v3f[...])
        fu = jnp.exp(lax.dot_general(u2n, u3n_all, (((1,), (1,)), ((), ())),
                                     preferred_element_type=jnp.float32)
                     / _TAU)
        fv = jnp.exp(lax.dot_general(v2n, v3n_all, (((1,), (1,)), ((), ())),
                                     preferred_element_type=jnp.float32)
                     / _TAU)
        du = jnp.exp(jnp.sum(u2n * u3n_blk, axis=1, keepdims=True) / _TAU)
        dv = jnp.exp(jnp.sum(v2n * v3n_blk, axis=1, keepdims=True) / _TAU)
        diag_u[pl.ds(i * RB2, RB2), :] = du
        diag_v[pl.ds(i * RB2, RB2), :] = dv

        accs[0] = accs[0] + sb1
        accs[1] = accs[1] + r1
        accs[2] = accs[2] + sb2
        accs[3] = accs[3] + r2
        accs[4] = accs[4] + jnp.sum(fu)
        accs[5] = accs[5] + jnp.sum(fv)

        @pl.when(i == NB2 - 1)
        def _():
            du_all = diag_u[...]
            dv_all = diag_v[...]
            pos = du_all + dv_all
            neg = (accs[4] + accs[5]) - du_all - dv_all
            cl = -jnp.log(pos / (pos + neg))
            cl_mean = jnp.sum(cl) / float(_B)
            loss = (-accs[0] + _REG * accs[1]
                    + (-accs[2] / float(_B) + _REG * accs[3])
                    + cl_mean)
            out[0, 0] = loss

    return pl.pallas_call(
        body,
        grid=(NB2,),
        in_specs=[bspec] * 6 + [nspec, nspec, wspec, fullspec, fullspec],
        out_specs=pl.BlockSpec(memory_space=pltpu.SMEM),
        out_shape=jax.ShapeDtypeStruct((1, 1), jnp.float32),
        scratch_shapes=[
            pltpu.VMEM((_B, 1), jnp.float32),
            pltpu.VMEM((_B, 1), jnp.float32),
            pltpu.SMEM((8,), jnp.float32),
        ],
    )(u1, v1, u2, v2, u3, v3, n1, n2, wsg, u3, v3)


def kernel(u, v, w, n, edge_index_p, edge_index_n, edge_index_n1,
           E_pos, E_neg, E_item, E_item_n, E2,
           W0, b0, W1, b1, Wa, ba, Wq):
    f32 = jnp.float32
    i32 = jnp.int32
    npad = _NP - _N
    pe = _NEP - _NE
    fill = (_N + (jnp.arange(pe, dtype=i32) % npad)).astype(i32)

    def pad_edges(ei):
        src = jnp.concatenate([ei[0].astype(i32), fill])
        dst = jnp.concatenate([ei[1].astype(i32), fill])
        return (src.reshape(_NEP // _CH, _CH), dst.reshape(_NEP // _CH, _CH))

    sp_, dp_ = pad_edges(edge_index_p)
    sn_, dn_ = pad_edges(edge_index_n)
    sm_, dm_ = pad_edges(edge_index_n1)
    idx6 = jnp.stack([sp_, dp_, sn_, dn_, sm_, dm_])

    dpart = _sc_degrees(idx6).reshape(6, _NW, _NP)

    zrows = jnp.zeros((npad, _D), f32)
    x0p = jnp.concatenate([E_pos.astype(f32), E_item.astype(f32), zrows])
    x0n = jnp.concatenate([E_neg.astype(f32), E_item_n.astype(f32), zrows])

    prep = _tc_prep(dpart, x0p, x0n)
    y0 = prep[:6]
    isd6 = prep[6]
    wsd3 = prep[7]
    srcs = jnp.stack([sp_, sn_, sm_])
    dsts = jnp.stack([dp_, dn_, dm_])
    l1 = _sc_gconv1(y0, srcs, dsts, wsd3)
    acc1 = l1[:6]
    z1 = l1[6:]
    accsum = _sc_gconv2(z1, srcs, dsts, acc1)

    e2p = jnp.concatenate([E2.astype(f32), zrows])
    zc, snt, sn1t = _tc_combine(isd6, x0p, x0n, e2p, accsum,
                                W0, b0.reshape(1, _D), W1, b1.reshape(1, _D),
                                Wa, ba.reshape(1, _D), Wq)

    u2d = u.astype(i32).reshape(_NW, _CH)
    v2d = v.astype(i32).reshape(_NW, _CH)
    n2d = n.astype(i32).reshape((_B * _K) // _CH, _CH)
    g = _sc_batch_gather(zc, snt, sn1t, u2d, v2d, n2d)
    u1g, v1g, n1g, u2g, v2g, n2g, u3g, v3g = g

    loss = _tc_loss(u1g, v1g, u2g, v2g, u3g, v3g,
                    n1g.reshape(_B, _K, _D), n2g.reshape(_B, _K, _D),
                    w.astype(f32).reshape(_B, 1))
    return loss[0, 0]
